# Initial kernel scaffold; baseline (speedup 1.0000x reference)
import jax
import jax.numpy as jnp
from jax.experimental import pallas as pl


def kernel(h, r, t, w, n_hn, n_rel_hn, n_t, n_h, n_rel_tn, n_tn, s_h, s_r, s_t, s_w, ent_real, ent_imag, rel, lin_w, lin_b):
    raise NotImplementedError("write your pallas kernel here")


# speedup vs baseline: 1.6216x; 1.6216x over previous
import jax
import jax.numpy as jnp
from jax import lax
from jax.experimental import pallas as pl
from jax.experimental.pallas import tpu as pltpu
from jax.experimental.pallas import tpu_sc as plsc

NUM_CONS = 1000000
NUM_RELS = 1000
DIM = 32
BATCH = 4096
NEG = 20
REG_SCALE = 0.0001
MARGIN = 2.0
EMB_RANGE = (MARGIN + 2.0) / DIM
PI = 3.141592653589793

L = 16
NC, NS = 2, 16
NW = NC * NS
BPW = BATCH // NW
NEG_PER_W = BPW * NEG
NEG2 = 2 * NEG_PER_W
CHUNK = 128
NCHUNK = NEG2 // CHUNK
GROUPS = CHUNK // L


def _sigmoid(z):
    return 1.0 / (1.0 + jnp.exp(-z))


def _sc_body(h_hbm, t_hbm, r_hbm, w_hbm,
             a_hn_hbm, a_nh_hbm, b_nt_hbm, b_tn_hbm, r_hn_hbm, r_tn_hbm,
             ent_r_hbm, ent_i_hbm, cos_hbm, sin_hbm, ab_hbm,
             out_hbm,
             cos_v, sin_v, aidx_v, bidx_v, ridx_v,
             hidx_v, tidx_v, pridx_v, w_v, ab_v,
             e0, e1, e2, e3, srow, out_v, sem):
    wid = lax.axis_index("s") * NC + lax.axis_index("c")
    base = wid * BPW
    nb = wid * NEG_PER_W

    stage = [
        (cos_hbm, cos_v),
        (sin_hbm, sin_v),
        (a_hn_hbm.at[pl.ds(nb, NEG_PER_W)], aidx_v.at[pl.ds(0, NEG_PER_W)]),
        (a_nh_hbm.at[pl.ds(nb, NEG_PER_W)], aidx_v.at[pl.ds(NEG_PER_W, NEG_PER_W)]),
        (b_nt_hbm.at[pl.ds(nb, NEG_PER_W)], bidx_v.at[pl.ds(0, NEG_PER_W)]),
        (b_tn_hbm.at[pl.ds(nb, NEG_PER_W)], bidx_v.at[pl.ds(NEG_PER_W, NEG_PER_W)]),
        (r_hn_hbm.at[pl.ds(nb, NEG_PER_W)], ridx_v.at[pl.ds(0, NEG_PER_W)]),
        (r_tn_hbm.at[pl.ds(nb, NEG_PER_W)], ridx_v.at[pl.ds(NEG_PER_W, NEG_PER_W)]),
        (h_hbm.at[pl.ds(base, BPW)], hidx_v),
        (t_hbm.at[pl.ds(base, BPW)], tidx_v),
        (r_hbm.at[pl.ds(base, BPW)], pridx_v),
        (w_hbm.at[pl.ds(base, BPW)], w_v),
        (ab_hbm, ab_v),
    ]
    handles = [pltpu.async_copy(s, d, sem) for s, d in stage]
    for hnd in handles:
        hnd.wait()

    av = ab_v[pl.ds(0, L)]
    bv = ab_v[pl.ds(L, L)]
    zeros = jnp.zeros((L,), jnp.float32)
    lastmask = lax.iota(jnp.int32, L) == (L - 1)

    def score16(rowbase, rel_ref, rel_off, wants_reg, regacc):
        rvec = rel_ref[pl.ds(rel_off + rowbase, L)]
        for i in range(L):
            rb = rowbase + i
            hr0 = e0[rb, pl.ds(0, L)]
            hr1 = e0[rb, pl.ds(L, L)]
            hi0 = e1[rb, pl.ds(0, L)]
            hi1 = e1[rb, pl.ds(L, L)]
            tr0 = e2[rb, pl.ds(0, L)]
            tr1 = e2[rb, pl.ds(L, L)]
            ti0 = e3[rb, pl.ds(0, L)]
            ti1 = e3[rb, pl.ds(L, L)]
            rid = rvec[i]
            toff = rid * DIM
            cc0 = cos_v[pl.ds(toff, L)]
            cc1 = cos_v[pl.ds(toff + L, L)]
            ss0 = sin_v[pl.ds(toff, L)]
            ss1 = sin_v[pl.ds(toff + L, L)]
            u0 = hr0 * cc0 - hi0 * ss0 - tr0
            v0 = hr0 * ss0 + hi0 * cc0 - ti0
            u1 = hr1 * cc1 - hi1 * ss1 - tr1
            v1 = hr1 * ss1 + hi1 * cc1 - ti1
            acc = (u0 * u0 + v0 * v0) + (u1 * u1 + v1 * v1)
            c = plsc.cumsum(acc)
            plsc.store_scatter(srow, [jnp.full((L,), i, jnp.int32)], c,
                               mask=lastmask)
            if wants_reg:
                regacc = regacc + (hr0 * hr0 + hr1 * hr1 + hi0 * hi0
                                   + hi1 * hi1 + tr0 * tr0 + tr1 * tr1
                                   + ti0 * ti0 + ti1 * ti1 + cc0 * cc0
                                   + cc1 * cc1 + ss0 * ss0 + ss1 * ss1)
        sv = srow[...]
        p = _sigmoid(av * (MARGIN - sv) + bv)
        return p, regacc

    pos = [
        pltpu.async_copy(ent_r_hbm.at[hidx_v], e0, sem),
        pltpu.async_copy(ent_i_hbm.at[hidx_v], e1, sem),
        pltpu.async_copy(ent_r_hbm.at[tidx_v], e2, sem),
        pltpu.async_copy(ent_i_hbm.at[tidx_v], e3, sem),
    ]
    for hnd in pos:
        hnd.wait()

    def pos_group(g, carry):
        pacc, regacc = carry
        p, regacc = score16(g * L, pridx_v, 0, True, regacc)
        wv = w_v[pl.ds(g * L, L)]
        dlt = p - wv
        return pacc + dlt * dlt, regacc

    pacc, regacc = lax.fori_loop(0, GROUPS, pos_group, (zeros, zeros))

    def neg_chunk(c, nacc):
        avw = aidx_v.at[pl.ds(c * CHUNK, CHUNK)]
        bvw = bidx_v.at[pl.ds(c * CHUNK, CHUNK)]
        gs = [
            pltpu.async_copy(ent_r_hbm.at[avw], e0, sem),
            pltpu.async_copy(ent_i_hbm.at[avw], e1, sem),
            pltpu.async_copy(ent_r_hbm.at[bvw], e2, sem),
            pltpu.async_copy(ent_i_hbm.at[bvw], e3, sem),
        ]
        for hnd in gs:
            hnd.wait()

        def grp(g, acc):
            p, _ = score16(g * L, ridx_v, c * CHUNK, False, None)
            return acc + p * p

        return lax.fori_loop(0, GROUPS, grp, nacc)

    nacc = lax.fori_loop(0, NCHUNK, neg_chunk, zeros)

    part = (pacc * (1.0 / BATCH)
            + nacc * (1.0 / (2.0 * NEG * BATCH))
            + regacc * (REG_SCALE / (2.0 * BATCH)))
    out_v[...] = part
    pltpu.sync_copy(out_v, out_hbm.at[pl.ds(wid * L, L)])


def _trig_body(rel_ref, cos_ref, sin_ref):
    x = rel_ref[...] * (PI / EMB_RANGE)
    cos_ref[...] = jnp.cos(x)
    sin_ref[...] = jnp.sin(x)


def _fin_body(p_ref, o_ref):
    o_ref[0, 0] = jnp.sum(p_ref[...])


def kernel(h, r, t, w, n_hn, n_rel_hn, n_t, n_h, n_rel_tn, n_tn,
           s_h, s_r, s_t, s_w, ent_real, ent_imag, rel, lin_w, lin_b):
    del s_h, s_r, s_t, s_w
    i32 = jnp.int32
    cos_t, sin_t = pl.pallas_call(
        _trig_body,
        out_shape=(jax.ShapeDtypeStruct((NUM_RELS, DIM), jnp.float32),
                   jax.ShapeDtypeStruct((NUM_RELS, DIM), jnp.float32)),
    )(rel)

    ab = jnp.concatenate([
        jnp.broadcast_to(lin_w.reshape(()), (L,)),
        jnp.broadcast_to(lin_b.reshape(()), (L,)),
    ]).astype(jnp.float32)

    mesh = plsc.VectorSubcoreMesh(core_axis_name="c", subcore_axis_name="s",
                                  num_cores=NC, num_subcores=NS)
    partials = pl.kernel(
        _sc_body,
        out_type=jax.ShapeDtypeStruct((NW * L,), jnp.float32),
        mesh=mesh,
        compiler_params=pltpu.CompilerParams(needs_layout_passes=False,
                                             use_tc_tiling_on_sc=False),
        scratch_types=[
            pltpu.VMEM((NUM_RELS * DIM,), jnp.float32),
            pltpu.VMEM((NUM_RELS * DIM,), jnp.float32),
            pltpu.VMEM((NEG2,), i32),
            pltpu.VMEM((NEG2,), i32),
            pltpu.VMEM((NEG2,), i32),
            pltpu.VMEM((BPW,), i32),
            pltpu.VMEM((BPW,), i32),
            pltpu.VMEM((BPW,), i32),
            pltpu.VMEM((BPW,), jnp.float32),
            pltpu.VMEM((2 * L,), jnp.float32),
            pltpu.VMEM((CHUNK, DIM), jnp.float32),
            pltpu.VMEM((CHUNK, DIM), jnp.float32),
            pltpu.VMEM((CHUNK, DIM), jnp.float32),
            pltpu.VMEM((CHUNK, DIM), jnp.float32),
            pltpu.VMEM((L,), jnp.float32),
            pltpu.VMEM((L,), jnp.float32),
            pltpu.SemaphoreType.DMA,
        ],
    )(
        h.astype(i32), t.astype(i32), r.astype(i32), w.astype(jnp.float32),
        n_hn.reshape(-1).astype(i32), n_h.reshape(-1).astype(i32),
        n_t.reshape(-1).astype(i32), n_tn.reshape(-1).astype(i32),
        n_rel_hn.reshape(-1).astype(i32), n_rel_tn.reshape(-1).astype(i32),
        ent_real, ent_imag,
        cos_t.reshape(-1), sin_t.reshape(-1), ab,
    )

    loss = pl.pallas_call(
        _fin_body,
        out_shape=jax.ShapeDtypeStruct((1, 1), jnp.float32),
        out_specs=pl.BlockSpec(memory_space=pltpu.SMEM),
    )(partials.reshape(NW, L))
    return loss[0, 0]


# speedup vs baseline: 1.9767x; 1.2190x over previous
import jax
import jax.numpy as jnp
from jax import lax
from jax.experimental import pallas as pl
from jax.experimental.pallas import tpu as pltpu
from jax.experimental.pallas import tpu_sc as plsc

NUM_CONS = 1000000
NUM_RELS = 1000
DIM = 32
BATCH = 4096
NEG = 20
REG_SCALE = 0.0001
MARGIN = 2.0
EMB_RANGE = (MARGIN + 2.0) / DIM
PI = 3.141592653589793

L = 16
NC, NS = 2, 16
NW = NC * NS
BPW = BATCH // NW
NEG_PER_W = BPW * NEG
NEG2 = 2 * NEG_PER_W
CHUNK = 64
NCHUNK = NEG2 // CHUNK
GROUPS = CHUNK // L
PREP_B = 2048
PREP_G = 123
QROWS = PREP_B * PREP_G


def _sigmoid(z):
    return 1.0 / (1.0 + jnp.exp(-z))


def _sc_body(h_hbm, t_hbm, r_hbm, w_hbm,
             a_hn_hbm, a_nh_hbm, b_nt_hbm, b_tn_hbm, r_hn_hbm, r_tn_hbm,
             as_hn_hbm, as_nh_hbm, bs_nt_hbm, bs_tn_hbm, hs_hbm, ts_hbm,
             ent_r_hbm, ent_i_hbm, cos_hbm, sin_hbm, ab_hbm,
             out_hbm,
             cos_v, sin_v, aidx_v, bidx_v, ridx_v, asid_v, bsid_v,
             hidx_v, tidx_v, hs_v, ts_v, pridx_v, w_v, ab_v,
             e0, e1, e2, e3, srow, out_v, sem):
    wid = lax.axis_index("s") * NC + lax.axis_index("c")
    base = wid * BPW
    nb = wid * NEG_PER_W

    stage = [
        (cos_hbm, cos_v),
        (sin_hbm, sin_v),
        (a_hn_hbm.at[pl.ds(nb, NEG_PER_W)], aidx_v.at[pl.ds(0, NEG_PER_W)]),
        (a_nh_hbm.at[pl.ds(nb, NEG_PER_W)], aidx_v.at[pl.ds(NEG_PER_W, NEG_PER_W)]),
        (b_nt_hbm.at[pl.ds(nb, NEG_PER_W)], bidx_v.at[pl.ds(0, NEG_PER_W)]),
        (b_tn_hbm.at[pl.ds(nb, NEG_PER_W)], bidx_v.at[pl.ds(NEG_PER_W, NEG_PER_W)]),
        (r_hn_hbm.at[pl.ds(nb, NEG_PER_W)], ridx_v.at[pl.ds(0, NEG_PER_W)]),
        (r_tn_hbm.at[pl.ds(nb, NEG_PER_W)], ridx_v.at[pl.ds(NEG_PER_W, NEG_PER_W)]),
        (as_hn_hbm.at[pl.ds(nb, NEG_PER_W)], asid_v.at[pl.ds(0, NEG_PER_W)]),
        (as_nh_hbm.at[pl.ds(nb, NEG_PER_W)], asid_v.at[pl.ds(NEG_PER_W, NEG_PER_W)]),
        (bs_nt_hbm.at[pl.ds(nb, NEG_PER_W)], bsid_v.at[pl.ds(0, NEG_PER_W)]),
        (bs_tn_hbm.at[pl.ds(nb, NEG_PER_W)], bsid_v.at[pl.ds(NEG_PER_W, NEG_PER_W)]),
        (h_hbm.at[pl.ds(base, BPW)], hidx_v),
        (t_hbm.at[pl.ds(base, BPW)], tidx_v),
        (hs_hbm.at[pl.ds(base, BPW)], hs_v),
        (ts_hbm.at[pl.ds(base, BPW)], ts_v),
        (r_hbm.at[pl.ds(base, BPW)], pridx_v),
        (w_hbm.at[pl.ds(base, BPW)], w_v),
        (ab_hbm, ab_v),
    ]
    handles = [pltpu.async_copy(s, d, sem) for s, d in stage]
    for hnd in handles:
        hnd.wait()

    av = ab_v[pl.ds(0, L)]
    bv = ab_v[pl.ds(L, L)]
    zeros = jnp.zeros((L,), jnp.float32)
    lastmask = lax.iota(jnp.int32, L) == (L - 1)

    def score16(rowbase, avec, bvec, rvec, wants_reg, regacc):
        for i in range(L):
            rb = rowbase + i
            ca = avec[i] * DIM
            cb = bvec[i] * DIM
            hr0 = e0[rb, pl.ds(ca, L)]
            hr1 = e0[rb, pl.ds(ca + L, L)]
            hi0 = e1[rb, pl.ds(ca, L)]
            hi1 = e1[rb, pl.ds(ca + L, L)]
            tr0 = e2[rb, pl.ds(cb, L)]
            tr1 = e2[rb, pl.ds(cb + L, L)]
            ti0 = e3[rb, pl.ds(cb, L)]
            ti1 = e3[rb, pl.ds(cb + L, L)]
            rid = rvec[i]
            toff = rid * DIM
            cc0 = cos_v[pl.ds(toff, L)]
            cc1 = cos_v[pl.ds(toff + L, L)]
            ss0 = sin_v[pl.ds(toff, L)]
            ss1 = sin_v[pl.ds(toff + L, L)]
            u0 = hr0 * cc0 - hi0 * ss0 - tr0
            v0 = hr0 * ss0 + hi0 * cc0 - ti0
            u1 = hr1 * cc1 - hi1 * ss1 - tr1
            v1 = hr1 * ss1 + hi1 * cc1 - ti1
            acc = (u0 * u0 + v0 * v0) + (u1 * u1 + v1 * v1)
            c = plsc.cumsum(acc)
            plsc.store_scatter(srow, [jnp.full((L,), i, jnp.int32)], c,
                               mask=lastmask)
            if wants_reg:
                regacc = regacc + (hr0 * hr0 + hr1 * hr1 + hi0 * hi0
                                   + hi1 * hi1 + tr0 * tr0 + tr1 * tr1
                                   + ti0 * ti0 + ti1 * ti1 + cc0 * cc0
                                   + cc1 * cc1 + ss0 * ss0 + ss1 * ss1)
        sv = srow[...]
        p = _sigmoid(av * (MARGIN - sv) + bv)
        return p, regacc

    pacc = zeros
    regacc = zeros
    for k in range(2):
        gs = [
            pltpu.async_copy(ent_r_hbm.at[hs_v.at[pl.ds(k * CHUNK, CHUNK)]], e0, sem),
            pltpu.async_copy(ent_i_hbm.at[hs_v.at[pl.ds(k * CHUNK, CHUNK)]], e1, sem),
            pltpu.async_copy(ent_r_hbm.at[ts_v.at[pl.ds(k * CHUNK, CHUNK)]], e2, sem),
            pltpu.async_copy(ent_i_hbm.at[ts_v.at[pl.ds(k * CHUNK, CHUNK)]], e3, sem),
        ]
        for hnd in gs:
            hnd.wait()

        def pos_group(g, carry, _k=k):
            pacc, regacc = carry
            o = _k * CHUNK + g * L
            p, regacc = score16(g * L, hidx_v[pl.ds(o, L)],
                                tidx_v[pl.ds(o, L)], pridx_v[pl.ds(o, L)],
                                True, regacc)
            wv = w_v[pl.ds(o, L)]
            dlt = p - wv
            return pacc + dlt * dlt, regacc

        pacc, regacc = lax.fori_loop(0, GROUPS, pos_group, (pacc, regacc))

    def neg_chunk(c, nacc):
        avw = asid_v.at[pl.ds(c * CHUNK, CHUNK)]
        bvw = bsid_v.at[pl.ds(c * CHUNK, CHUNK)]
        gs = [
            pltpu.async_copy(ent_r_hbm.at[avw], e0, sem),
            pltpu.async_copy(ent_i_hbm.at[avw], e1, sem),
            pltpu.async_copy(ent_r_hbm.at[bvw], e2, sem),
            pltpu.async_copy(ent_i_hbm.at[bvw], e3, sem),
        ]
        for hnd in gs:
            hnd.wait()

        def grp(g, acc):
            o = c * CHUNK + g * L
            p, _ = score16(g * L, aidx_v[pl.ds(o, L)], bidx_v[pl.ds(o, L)],
                           ridx_v[pl.ds(o, L)], False, None)
            return acc + p * p

        return lax.fori_loop(0, GROUPS, grp, nacc)

    nacc = lax.fori_loop(0, NCHUNK, neg_chunk, zeros)

    part = (pacc * (1.0 / BATCH)
            + nacc * (1.0 / (2.0 * NEG * BATCH))
            + regacc * (REG_SCALE / (2.0 * BATCH)))
    out_v[...] = part
    pltpu.sync_copy(out_v, out_hbm.at[pl.ds(wid * L, L)])


def _trig_body(rel_ref, cos_ref, sin_ref):
    x = rel_ref[...] * (PI / EMB_RANGE)
    cos_ref[...] = jnp.cos(x)
    sin_ref[...] = jnp.sin(x)


def _prep_body(r0, r1, r2, r3, i0, i1, i2, i3, or_ref, oi_ref):
    or_ref[...] = jnp.concatenate(
        [r0[...].T, r1[...].T, r2[...].T, r3[...].T], axis=1)
    oi_ref[...] = jnp.concatenate(
        [i0[...].T, i1[...].T, i2[...].T, i3[...].T], axis=1)


def _fin_body(p_ref, o_ref):
    o_ref[0, 0] = jnp.sum(p_ref[...])


def kernel(h, r, t, w, n_hn, n_rel_hn, n_t, n_h, n_rel_tn, n_tn,
           s_h, s_r, s_t, s_w, ent_real, ent_imag, rel, lin_w, lin_b):
    del s_h, s_r, s_t, s_w
    i32 = jnp.int32
    cos_t, sin_t = pl.pallas_call(
        _trig_body,
        out_shape=(jax.ShapeDtypeStruct((NUM_RELS, DIM), jnp.float32),
                   jax.ShapeDtypeStruct((NUM_RELS, DIM), jnp.float32)),
    )(rel)

    hh = h.astype(i32)
    tt = t.astype(i32)
    a_flat = n_hn.reshape(-1).astype(i32)
    nh_flat = n_h.reshape(-1).astype(i32)
    b_flat = n_t.reshape(-1).astype(i32)
    ntn_flat = n_tn.reshape(-1).astype(i32)
    ab = jnp.concatenate([
        jnp.broadcast_to(lin_w.reshape(()), (L,)),
        jnp.broadcast_to(lin_b.reshape(()), (L,)),
    ]).astype(jnp.float32)

    maxb = NUM_CONS // PREP_B
    qspecs = [pl.BlockSpec((DIM, PREP_B),
                           lambda c, _j=j: (0, jnp.minimum(_j * PREP_G + c, maxb)))
              for j in range(4)]
    er4, ei4 = pl.pallas_call(
        _prep_body,
        grid=(PREP_G,),
        in_specs=qspecs + qspecs,
        out_specs=[pl.BlockSpec((PREP_B, 4 * DIM), lambda c: (c, 0)),
                   pl.BlockSpec((PREP_B, 4 * DIM), lambda c: (c, 0))],
        out_shape=(jax.ShapeDtypeStruct((QROWS, 4 * DIM), jnp.float32),
                   jax.ShapeDtypeStruct((QROWS, 4 * DIM), jnp.float32)),
    )(*([ent_real.T] * 4), *([ent_imag.T] * 4))

    mesh = plsc.VectorSubcoreMesh(core_axis_name="c", subcore_axis_name="s",
                                  num_cores=NC, num_subcores=NS)
    partials = pl.kernel(
        _sc_body,
        out_type=jax.ShapeDtypeStruct((NW * L,), jnp.float32),
        mesh=mesh,
        compiler_params=pltpu.CompilerParams(needs_layout_passes=False,
                                             use_tc_tiling_on_sc=True),
        scratch_types=[
            pltpu.VMEM((NUM_RELS * DIM,), jnp.float32),
            pltpu.VMEM((NUM_RELS * DIM,), jnp.float32),
            pltpu.VMEM((NEG2,), i32),
            pltpu.VMEM((NEG2,), i32),
            pltpu.VMEM((NEG2,), i32),
            pltpu.VMEM((NEG2,), i32),
            pltpu.VMEM((NEG2,), i32),
            pltpu.VMEM((BPW,), i32),
            pltpu.VMEM((BPW,), i32),
            pltpu.VMEM((BPW,), i32),
            pltpu.VMEM((BPW,), i32),
            pltpu.VMEM((BPW,), i32),
            pltpu.VMEM((BPW,), jnp.float32),
            pltpu.VMEM((2 * L,), jnp.float32),
            pltpu.VMEM((CHUNK, 4 * DIM), jnp.float32),
            pltpu.VMEM((CHUNK, 4 * DIM), jnp.float32),
            pltpu.VMEM((CHUNK, 4 * DIM), jnp.float32),
            pltpu.VMEM((CHUNK, 4 * DIM), jnp.float32),
            pltpu.VMEM((L,), jnp.float32),
            pltpu.VMEM((L,), jnp.float32),
            pltpu.SemaphoreType.DMA,
        ],
    )(
        hh // QROWS, tt // QROWS, r.astype(i32), w.astype(jnp.float32),
        a_flat // QROWS, nh_flat // QROWS, b_flat // QROWS, ntn_flat // QROWS,
        n_rel_hn.reshape(-1).astype(i32), n_rel_tn.reshape(-1).astype(i32),
        a_flat % QROWS, nh_flat % QROWS, b_flat % QROWS, ntn_flat % QROWS,
        hh % QROWS, tt % QROWS,
        er4, ei4,
        cos_t.reshape(-1), sin_t.reshape(-1), ab,
    )

    loss = pl.pallas_call(
        _fin_body,
        out_shape=jax.ShapeDtypeStruct((1, 1), jnp.float32),
        out_specs=pl.BlockSpec(memory_space=pltpu.SMEM),
    )(partials.reshape(NW, L))
    return loss[0, 0]


# speedup vs baseline: 2.2363x; 1.1314x over previous
import jax
import jax.numpy as jnp
from jax import lax
from jax.experimental import pallas as pl
from jax.experimental.pallas import tpu as pltpu
from jax.experimental.pallas import tpu_sc as plsc

NUM_CONS = 1000000
NUM_RELS = 1000
DIM = 32
BATCH = 4096
NEG = 20
REG_SCALE = 0.0001
MARGIN = 2.0
EMB_RANGE = (MARGIN + 2.0) / DIM
PI = 3.141592653589793

L = 16
NC, NS = 2, 16
NW = NC * NS
BPW = BATCH // NW
NEG_PER_W = BPW * NEG
NEG2 = 2 * NEG_PER_W
CHUNK = 32
NCHUNK = NEG2 // CHUNK
GROUPS = CHUNK // L
PREP_B = 2048
PREP_G = 123
QROWS = PREP_B * PREP_G


def _sigmoid(z):
    return 1.0 / (1.0 + jnp.exp(-z))


def _sc_body(h_hbm, t_hbm, r_hbm, w_hbm,
             a_hn_hbm, a_nh_hbm, b_nt_hbm, b_tn_hbm, r_hn_hbm, r_tn_hbm,
             as_hn_hbm, as_nh_hbm, bs_nt_hbm, bs_tn_hbm, hs_hbm, ts_hbm,
             ent_r_hbm, ent_i_hbm, cos_hbm, sin_hbm, ab_hbm,
             out_hbm,
             cos_v, sin_v, aidx_v, bidx_v, ridx_v, asid_v, bsid_v,
             hidx_v, tidx_v, hs_v, ts_v, pridx_v, w_v, ab_v,
             e0, e1, e2, e3, f0, f1, f2, f3, srow, out_v, sem, sem2):
    wid = lax.axis_index("s") * NC + lax.axis_index("c")
    base = wid * BPW
    nb = wid * NEG_PER_W

    stage = [
        (cos_hbm, cos_v),
        (sin_hbm, sin_v),
        (a_hn_hbm.at[pl.ds(nb, NEG_PER_W)], aidx_v.at[pl.ds(0, NEG_PER_W)]),
        (a_nh_hbm.at[pl.ds(nb, NEG_PER_W)], aidx_v.at[pl.ds(NEG_PER_W, NEG_PER_W)]),
        (b_nt_hbm.at[pl.ds(nb, NEG_PER_W)], bidx_v.at[pl.ds(0, NEG_PER_W)]),
        (b_tn_hbm.at[pl.ds(nb, NEG_PER_W)], bidx_v.at[pl.ds(NEG_PER_W, NEG_PER_W)]),
        (r_hn_hbm.at[pl.ds(nb, NEG_PER_W)], ridx_v.at[pl.ds(0, NEG_PER_W)]),
        (r_tn_hbm.at[pl.ds(nb, NEG_PER_W)], ridx_v.at[pl.ds(NEG_PER_W, NEG_PER_W)]),
        (as_hn_hbm.at[pl.ds(nb, NEG_PER_W)], asid_v.at[pl.ds(0, NEG_PER_W)]),
        (as_nh_hbm.at[pl.ds(nb, NEG_PER_W)], asid_v.at[pl.ds(NEG_PER_W, NEG_PER_W)]),
        (bs_nt_hbm.at[pl.ds(nb, NEG_PER_W)], bsid_v.at[pl.ds(0, NEG_PER_W)]),
        (bs_tn_hbm.at[pl.ds(nb, NEG_PER_W)], bsid_v.at[pl.ds(NEG_PER_W, NEG_PER_W)]),
        (h_hbm.at[pl.ds(base, BPW)], hidx_v),
        (t_hbm.at[pl.ds(base, BPW)], tidx_v),
        (hs_hbm.at[pl.ds(base, BPW)], hs_v),
        (ts_hbm.at[pl.ds(base, BPW)], ts_v),
        (r_hbm.at[pl.ds(base, BPW)], pridx_v),
        (w_hbm.at[pl.ds(base, BPW)], w_v),
        (ab_hbm, ab_v),
    ]
    handles = [pltpu.async_copy(s, d, sem) for s, d in stage]
    for hnd in handles:
        hnd.wait()

    av = ab_v[pl.ds(0, L)]
    bv = ab_v[pl.ds(L, L)]
    zeros = jnp.zeros((L,), jnp.float32)
    lastmask = lax.iota(jnp.int32, L) == (L - 1)

    def score16(bufs, rowbase, avec, bvec, rvec, wants_reg, regacc):
        b0, b1, b2, b3 = bufs
        for i in range(L):
            rb = rowbase + i
            ca = avec[i] * DIM
            cb = bvec[i] * DIM
            hr0 = b0[rb, pl.ds(ca, L)]
            hr1 = b0[rb, pl.ds(ca + L, L)]
            hi0 = b1[rb, pl.ds(ca, L)]
            hi1 = b1[rb, pl.ds(ca + L, L)]
            tr0 = b2[rb, pl.ds(cb, L)]
            tr1 = b2[rb, pl.ds(cb + L, L)]
            ti0 = b3[rb, pl.ds(cb, L)]
            ti1 = b3[rb, pl.ds(cb + L, L)]
            rid = rvec[i]
            toff = rid * DIM
            cc0 = cos_v[pl.ds(toff, L)]
            cc1 = cos_v[pl.ds(toff + L, L)]
            ss0 = sin_v[pl.ds(toff, L)]
            ss1 = sin_v[pl.ds(toff + L, L)]
            u0 = hr0 * cc0 - hi0 * ss0 - tr0
            v0 = hr0 * ss0 + hi0 * cc0 - ti0
            u1 = hr1 * cc1 - hi1 * ss1 - tr1
            v1 = hr1 * ss1 + hi1 * cc1 - ti1
            acc = (u0 * u0 + v0 * v0) + (u1 * u1 + v1 * v1)
            c = plsc.cumsum(acc)
            plsc.store_scatter(srow, [jnp.full((L,), i, jnp.int32)], c,
                               mask=lastmask)
            if wants_reg:
                regacc = regacc + (hr0 * hr0 + hr1 * hr1 + hi0 * hi0
                                   + hi1 * hi1 + tr0 * tr0 + tr1 * tr1
                                   + ti0 * ti0 + ti1 * ti1 + cc0 * cc0
                                   + cc1 * cc1 + ss0 * ss0 + ss1 * ss1)
        sv = srow[...]
        p = _sigmoid(av * (MARGIN - sv) + bv)
        return p, regacc

    pacc = zeros
    regacc = zeros
    for k in range(4):
        gs = [
            pltpu.async_copy(ent_r_hbm.at[hs_v.at[pl.ds(k * CHUNK, CHUNK)]], e0, sem),
            pltpu.async_copy(ent_i_hbm.at[hs_v.at[pl.ds(k * CHUNK, CHUNK)]], e1, sem),
            pltpu.async_copy(ent_r_hbm.at[ts_v.at[pl.ds(k * CHUNK, CHUNK)]], e2, sem),
            pltpu.async_copy(ent_i_hbm.at[ts_v.at[pl.ds(k * CHUNK, CHUNK)]], e3, sem),
        ]
        for hnd in gs:
            hnd.wait()

        def pos_group(g, carry, _k=k):
            pacc, regacc = carry
            o = _k * CHUNK + g * L
            p, regacc = score16((e0, e1, e2, e3), g * L, hidx_v[pl.ds(o, L)],
                                tidx_v[pl.ds(o, L)], pridx_v[pl.ds(o, L)],
                                True, regacc)
            wv = w_v[pl.ds(o, L)]
            dlt = p - wv
            return pacc + dlt * dlt, regacc

        pacc, regacc = lax.fori_loop(0, GROUPS, pos_group, (pacc, regacc))

    slots = ((e0, e1, e2, e3, sem), (f0, f1, f2, f3, sem2))

    def fire(c, slot):
        b0, b1, b2, b3, sm = slot
        avw = asid_v.at[pl.ds(c * CHUNK, CHUNK)]
        bvw = bsid_v.at[pl.ds(c * CHUNK, CHUNK)]
        pltpu.async_copy(ent_r_hbm.at[avw], b0, sm)
        pltpu.async_copy(ent_i_hbm.at[avw], b1, sm)
        pltpu.async_copy(ent_r_hbm.at[bvw], b2, sm)
        pltpu.async_copy(ent_i_hbm.at[bvw], b3, sm)

    def drain(slot):
        b0, b1, b2, b3, sm = slot
        for b in (b0, b1, b2, b3):
            pltpu.make_async_copy(ent_r_hbm.at[pl.ds(0, CHUNK)], b, sm).wait()

    def compute(c, slot, nacc):
        def grp(g, acc):
            o = c * CHUNK + g * L
            p, _ = score16(slot[:4], g * L, aidx_v[pl.ds(o, L)],
                           bidx_v[pl.ds(o, L)], ridx_v[pl.ds(o, L)],
                           False, None)
            return acc + p * p
        return lax.fori_loop(0, GROUPS, grp, nacc)

    fire(0, slots[0])
    fire(1, slots[1])

    def neg_pair(c2, nacc):
        c = c2 * 2
        drain(slots[0])
        nacc = compute(c, slots[0], nacc)

        @pl.when(c + 2 < NCHUNK)
        def _():
            fire(c + 2, slots[0])

        drain(slots[1])
        nacc = compute(c + 1, slots[1], nacc)

        @pl.when(c + 3 < NCHUNK)
        def _():
            fire(c + 3, slots[1])

        return nacc

    nacc = lax.fori_loop(0, NCHUNK // 2, neg_pair, zeros)

    part = (pacc * (1.0 / BATCH)
            + nacc * (1.0 / (2.0 * NEG * BATCH))
            + regacc * (REG_SCALE / (2.0 * BATCH)))
    out_v[...] = part
    pltpu.sync_copy(out_v, out_hbm.at[pl.ds(wid * L, L)])


def _trig_body(rel_ref, cos_ref, sin_ref):
    x = rel_ref[...] * (PI / EMB_RANGE)
    cos_ref[...] = jnp.cos(x)
    sin_ref[...] = jnp.sin(x)


def _prep_body(r0, r1, r2, r3, i0, i1, i2, i3, or_ref, oi_ref):
    or_ref[...] = jnp.concatenate(
        [r0[...].T, r1[...].T, r2[...].T, r3[...].T], axis=1)
    oi_ref[...] = jnp.concatenate(
        [i0[...].T, i1[...].T, i2[...].T, i3[...].T], axis=1)


def _fin_body(p_ref, o_ref):
    o_ref[0, 0] = jnp.sum(p_ref[...])


def kernel(h, r, t, w, n_hn, n_rel_hn, n_t, n_h, n_rel_tn, n_tn,
           s_h, s_r, s_t, s_w, ent_real, ent_imag, rel, lin_w, lin_b):
    del s_h, s_r, s_t, s_w
    i32 = jnp.int32
    cos_t, sin_t = pl.pallas_call(
        _trig_body,
        out_shape=(jax.ShapeDtypeStruct((NUM_RELS, DIM), jnp.float32),
                   jax.ShapeDtypeStruct((NUM_RELS, DIM), jnp.float32)),
    )(rel)

    hh = h.astype(i32)
    tt = t.astype(i32)
    a_flat = n_hn.reshape(-1).astype(i32)
    nh_flat = n_h.reshape(-1).astype(i32)
    b_flat = n_t.reshape(-1).astype(i32)
    ntn_flat = n_tn.reshape(-1).astype(i32)
    ab = jnp.concatenate([
        jnp.broadcast_to(lin_w.reshape(()), (L,)),
        jnp.broadcast_to(lin_b.reshape(()), (L,)),
    ]).astype(jnp.float32)

    maxb = NUM_CONS // PREP_B
    qspecs = [pl.BlockSpec((DIM, PREP_B),
                           lambda c, _j=j: (0, jnp.minimum(_j * PREP_G + c, maxb)))
              for j in range(4)]
    er4, ei4 = pl.pallas_call(
        _prep_body,
        grid=(PREP_G,),
        in_specs=qspecs + qspecs,
        out_specs=[pl.BlockSpec((PREP_B, 4 * DIM), lambda c: (c, 0)),
                   pl.BlockSpec((PREP_B, 4 * DIM), lambda c: (c, 0))],
        out_shape=(jax.ShapeDtypeStruct((QROWS, 4 * DIM), jnp.float32),
                   jax.ShapeDtypeStruct((QROWS, 4 * DIM), jnp.float32)),
    )(*([ent_real.T] * 4), *([ent_imag.T] * 4))

    mesh = plsc.VectorSubcoreMesh(core_axis_name="c", subcore_axis_name="s",
                                  num_cores=NC, num_subcores=NS)
    partials = pl.kernel(
        _sc_body,
        out_type=jax.ShapeDtypeStruct((NW * L,), jnp.float32),
        mesh=mesh,
        compiler_params=pltpu.CompilerParams(needs_layout_passes=False,
                                             use_tc_tiling_on_sc=True),
        scratch_types=[
            pltpu.VMEM((NUM_RELS * DIM,), jnp.float32),
            pltpu.VMEM((NUM_RELS * DIM,), jnp.float32),
            pltpu.VMEM((NEG2,), i32),
            pltpu.VMEM((NEG2,), i32),
            pltpu.VMEM((NEG2,), i32),
            pltpu.VMEM((NEG2,), i32),
            pltpu.VMEM((NEG2,), i32),
            pltpu.VMEM((BPW,), i32),
            pltpu.VMEM((BPW,), i32),
            pltpu.VMEM((BPW,), i32),
            pltpu.VMEM((BPW,), i32),
            pltpu.VMEM((BPW,), i32),
            pltpu.VMEM((BPW,), jnp.float32),
            pltpu.VMEM((2 * L,), jnp.float32),
            pltpu.VMEM((CHUNK, 4 * DIM), jnp.float32),
            pltpu.VMEM((CHUNK, 4 * DIM), jnp.float32),
            pltpu.VMEM((CHUNK, 4 * DIM), jnp.float32),
            pltpu.VMEM((CHUNK, 4 * DIM), jnp.float32),
            pltpu.VMEM((CHUNK, 4 * DIM), jnp.float32),
            pltpu.VMEM((CHUNK, 4 * DIM), jnp.float32),
            pltpu.VMEM((CHUNK, 4 * DIM), jnp.float32),
            pltpu.VMEM((CHUNK, 4 * DIM), jnp.float32),
            pltpu.VMEM((L,), jnp.float32),
            pltpu.VMEM((L,), jnp.float32),
            pltpu.SemaphoreType.DMA,
            pltpu.SemaphoreType.DMA,
        ],
    )(
        hh // QROWS, tt // QROWS, r.astype(i32), w.astype(jnp.float32),
        a_flat // QROWS, nh_flat // QROWS, b_flat // QROWS, ntn_flat // QROWS,
        n_rel_hn.reshape(-1).astype(i32), n_rel_tn.reshape(-1).astype(i32),
        a_flat % QROWS, nh_flat % QROWS, b_flat % QROWS, ntn_flat % QROWS,
        hh % QROWS, tt % QROWS,
        er4, ei4,
        cos_t.reshape(-1), sin_t.reshape(-1), ab,
    )

    loss = pl.pallas_call(
        _fin_body,
        out_shape=jax.ShapeDtypeStruct((1, 1), jnp.float32),
        out_specs=pl.BlockSpec(memory_space=pltpu.SMEM),
    )(partials.reshape(NW, L))
    return loss[0, 0]


# speedup vs baseline: 2.2640x; 1.0124x over previous
import jax
import jax.numpy as jnp
from jax import lax
from jax.experimental import pallas as pl
from jax.experimental.pallas import tpu as pltpu
from jax.experimental.pallas import tpu_sc as plsc

NUM_CONS = 1000000
NUM_RELS = 1000
DIM = 32
BATCH = 4096
NEG = 20
REG_SCALE = 0.0001
MARGIN = 2.0
EMB_RANGE = (MARGIN + 2.0) / DIM
PI = 3.141592653589793

L = 16
NC, NS = 2, 16
NW = NC * NS
BPW = BATCH // NW
NEG_PER_W = BPW * NEG
NEG2 = 2 * NEG_PER_W
CHUNK = 32
NCHUNK = NEG2 // CHUNK
GROUPS = CHUNK // L
PREP_B = 4096
PREP_G = 62
QROWS = PREP_B * PREP_G


def _sigmoid(z):
    return 1.0 / (1.0 + jnp.exp(-z))


def _sc_body(h_hbm, t_hbm, r_hbm, w_hbm,
             a_hn_hbm, a_nh_hbm, b_nt_hbm, b_tn_hbm, r_hn_hbm, r_tn_hbm,
             as_hn_hbm, as_nh_hbm, bs_nt_hbm, bs_tn_hbm, hs_hbm, ts_hbm,
             ent_r_hbm, ent_i_hbm, cos_hbm, sin_hbm, ab_hbm,
             out_hbm,
             cos_v, sin_v, aidx_v, bidx_v, ridx_v, asid_v, bsid_v,
             hidx_v, tidx_v, hs_v, ts_v, pridx_v, w_v, ab_v,
             e0, e1, e2, e3, f0, f1, f2, f3, srow, out_v, sem, sem2):
    wid = lax.axis_index("s") * NC + lax.axis_index("c")
    base = wid * BPW
    nb = wid * NEG_PER_W

    stage = [
        (cos_hbm, cos_v),
        (sin_hbm, sin_v),
        (a_hn_hbm.at[pl.ds(nb, NEG_PER_W)], aidx_v.at[pl.ds(0, NEG_PER_W)]),
        (a_nh_hbm.at[pl.ds(nb, NEG_PER_W)], aidx_v.at[pl.ds(NEG_PER_W, NEG_PER_W)]),
        (b_nt_hbm.at[pl.ds(nb, NEG_PER_W)], bidx_v.at[pl.ds(0, NEG_PER_W)]),
        (b_tn_hbm.at[pl.ds(nb, NEG_PER_W)], bidx_v.at[pl.ds(NEG_PER_W, NEG_PER_W)]),
        (r_hn_hbm.at[pl.ds(nb, NEG_PER_W)], ridx_v.at[pl.ds(0, NEG_PER_W)]),
        (r_tn_hbm.at[pl.ds(nb, NEG_PER_W)], ridx_v.at[pl.ds(NEG_PER_W, NEG_PER_W)]),
        (as_hn_hbm.at[pl.ds(nb, NEG_PER_W)], asid_v.at[pl.ds(0, NEG_PER_W)]),
        (as_nh_hbm.at[pl.ds(nb, NEG_PER_W)], asid_v.at[pl.ds(NEG_PER_W, NEG_PER_W)]),
        (bs_nt_hbm.at[pl.ds(nb, NEG_PER_W)], bsid_v.at[pl.ds(0, NEG_PER_W)]),
        (bs_tn_hbm.at[pl.ds(nb, NEG_PER_W)], bsid_v.at[pl.ds(NEG_PER_W, NEG_PER_W)]),
        (h_hbm.at[pl.ds(base, BPW)], hidx_v),
        (t_hbm.at[pl.ds(base, BPW)], tidx_v),
        (hs_hbm.at[pl.ds(base, BPW)], hs_v),
        (ts_hbm.at[pl.ds(base, BPW)], ts_v),
        (r_hbm.at[pl.ds(base, BPW)], pridx_v),
        (w_hbm.at[pl.ds(base, BPW)], w_v),
        (ab_hbm, ab_v),
    ]
    handles = [pltpu.async_copy(s, d, sem) for s, d in stage]
    for hnd in handles:
        hnd.wait()

    av = ab_v[pl.ds(0, L)]
    bv = ab_v[pl.ds(L, L)]
    zeros = jnp.zeros((L,), jnp.float32)
    lastmask = lax.iota(jnp.int32, L) == (L - 1)

    def score16(bufs, rowbase, avec, bvec, rvec, wants_reg, regacc):
        b0, b1, b2, b3 = bufs
        for i in range(L):
            rb = rowbase + i
            ca = avec[i] * DIM
            cb = bvec[i] * DIM
            hr0 = b0[rb, pl.ds(ca, L)]
            hr1 = b0[rb, pl.ds(ca + L, L)]
            hi0 = b1[rb, pl.ds(ca, L)]
            hi1 = b1[rb, pl.ds(ca + L, L)]
            tr0 = b2[rb, pl.ds(cb, L)]
            tr1 = b2[rb, pl.ds(cb + L, L)]
            ti0 = b3[rb, pl.ds(cb, L)]
            ti1 = b3[rb, pl.ds(cb + L, L)]
            rid = rvec[i]
            toff = rid * DIM
            cc0 = cos_v[pl.ds(toff, L)]
            cc1 = cos_v[pl.ds(toff + L, L)]
            ss0 = sin_v[pl.ds(toff, L)]
            ss1 = sin_v[pl.ds(toff + L, L)]
            u0 = hr0 * cc0 - hi0 * ss0 - tr0
            v0 = hr0 * ss0 + hi0 * cc0 - ti0
            u1 = hr1 * cc1 - hi1 * ss1 - tr1
            v1 = hr1 * ss1 + hi1 * cc1 - ti1
            acc = (u0 * u0 + v0 * v0) + (u1 * u1 + v1 * v1)
            c = plsc.cumsum(acc)
            plsc.store_scatter(srow, [jnp.full((L,), i, jnp.int32)], c,
                               mask=lastmask)
            if wants_reg:
                regacc = regacc + (hr0 * hr0 + hr1 * hr1 + hi0 * hi0
                                   + hi1 * hi1 + tr0 * tr0 + tr1 * tr1
                                   + ti0 * ti0 + ti1 * ti1 + cc0 * cc0
                                   + cc1 * cc1 + ss0 * ss0 + ss1 * ss1)
        sv = srow[...]
        p = _sigmoid(av * (MARGIN - sv) + bv)
        return p, regacc

    pacc = zeros
    regacc = zeros
    for k in range(4):
        gs = [
            pltpu.async_copy(ent_r_hbm.at[hs_v.at[pl.ds(k * CHUNK, CHUNK)]], e0, sem),
            pltpu.async_copy(ent_i_hbm.at[hs_v.at[pl.ds(k * CHUNK, CHUNK)]], e1, sem),
            pltpu.async_copy(ent_r_hbm.at[ts_v.at[pl.ds(k * CHUNK, CHUNK)]], e2, sem),
            pltpu.async_copy(ent_i_hbm.at[ts_v.at[pl.ds(k * CHUNK, CHUNK)]], e3, sem),
        ]
        for hnd in gs:
            hnd.wait()

        def pos_group(g, carry, _k=k):
            pacc, regacc = carry
            o = _k * CHUNK + g * L
            p, regacc = score16((e0, e1, e2, e3), g * L, hidx_v[pl.ds(o, L)],
                                tidx_v[pl.ds(o, L)], pridx_v[pl.ds(o, L)],
                                True, regacc)
            wv = w_v[pl.ds(o, L)]
            dlt = p - wv
            return pacc + dlt * dlt, regacc

        pacc, regacc = lax.fori_loop(0, GROUPS, pos_group, (pacc, regacc))

    slots = ((e0, e1, e2, e3, sem), (f0, f1, f2, f3, sem2))

    def fire(c, slot):
        b0, b1, b2, b3, sm = slot
        avw = asid_v.at[pl.ds(c * CHUNK, CHUNK)]
        bvw = bsid_v.at[pl.ds(c * CHUNK, CHUNK)]
        pltpu.async_copy(ent_r_hbm.at[avw], b0, sm)
        pltpu.async_copy(ent_i_hbm.at[avw], b1, sm)
        pltpu.async_copy(ent_r_hbm.at[bvw], b2, sm)
        pltpu.async_copy(ent_i_hbm.at[bvw], b3, sm)

    def drain(slot):
        b0, b1, b2, b3, sm = slot
        for b in (b0, b1, b2, b3):
            pltpu.make_async_copy(ent_r_hbm.at[pl.ds(0, CHUNK)], b, sm).wait()

    def compute(c, slot, nacc):
        def grp(g, acc):
            o = c * CHUNK + g * L
            p, _ = score16(slot[:4], g * L, aidx_v[pl.ds(o, L)],
                           bidx_v[pl.ds(o, L)], ridx_v[pl.ds(o, L)],
                           False, None)
            return acc + p * p
        return lax.fori_loop(0, GROUPS, grp, nacc)

    fire(0, slots[0])
    fire(1, slots[1])

    def neg_pair(c2, nacc):
        c = c2 * 2
        drain(slots[0])
        nacc = compute(c, slots[0], nacc)

        @pl.when(c + 2 < NCHUNK)
        def _():
            fire(c + 2, slots[0])

        drain(slots[1])
        nacc = compute(c + 1, slots[1], nacc)

        @pl.when(c + 3 < NCHUNK)
        def _():
            fire(c + 3, slots[1])

        return nacc

    nacc = lax.fori_loop(0, NCHUNK // 2, neg_pair, zeros)

    part = (pacc * (1.0 / BATCH)
            + nacc * (1.0 / (2.0 * NEG * BATCH))
            + regacc * (REG_SCALE / (2.0 * BATCH)))
    out_v[...] = part
    pltpu.sync_copy(out_v, out_hbm.at[pl.ds(wid * L, L)])


def _trig_body(rel_ref, cos_ref, sin_ref):
    x = rel_ref[...] * (PI / EMB_RANGE)
    cos_ref[...] = jnp.cos(x)
    sin_ref[...] = jnp.sin(x)


def _prep_body(r0, r1, r2, r3, i0, i1, i2, i3, or_ref, oi_ref):
    or_ref[...] = jnp.concatenate(
        [r0[...].T, r1[...].T, r2[...].T, r3[...].T], axis=1)
    oi_ref[...] = jnp.concatenate(
        [i0[...].T, i1[...].T, i2[...].T, i3[...].T], axis=1)


def _fin_body(p_ref, o_ref):
    o_ref[0, 0] = jnp.sum(p_ref[...])


def kernel(h, r, t, w, n_hn, n_rel_hn, n_t, n_h, n_rel_tn, n_tn,
           s_h, s_r, s_t, s_w, ent_real, ent_imag, rel, lin_w, lin_b):
    del s_h, s_r, s_t, s_w
    i32 = jnp.int32
    cos_t, sin_t = pl.pallas_call(
        _trig_body,
        out_shape=(jax.ShapeDtypeStruct((NUM_RELS, DIM), jnp.float32),
                   jax.ShapeDtypeStruct((NUM_RELS, DIM), jnp.float32)),
    )(rel)

    hh = h.astype(i32)
    tt = t.astype(i32)
    a_flat = n_hn.reshape(-1).astype(i32)
    nh_flat = n_h.reshape(-1).astype(i32)
    b_flat = n_t.reshape(-1).astype(i32)
    ntn_flat = n_tn.reshape(-1).astype(i32)
    ab = jnp.concatenate([
        jnp.broadcast_to(lin_w.reshape(()), (L,)),
        jnp.broadcast_to(lin_b.reshape(()), (L,)),
    ]).astype(jnp.float32)

    maxb = NUM_CONS // PREP_B
    qspecs = [pl.BlockSpec((DIM, PREP_B),
                           lambda c, _j=j: (0, jnp.minimum(_j * PREP_G + c, maxb)))
              for j in range(4)]
    er4, ei4 = pl.pallas_call(
        _prep_body,
        grid=(PREP_G,),
        in_specs=qspecs + qspecs,
        out_specs=[pl.BlockSpec((PREP_B, 4 * DIM), lambda c: (c, 0)),
                   pl.BlockSpec((PREP_B, 4 * DIM), lambda c: (c, 0))],
        out_shape=(jax.ShapeDtypeStruct((QROWS, 4 * DIM), jnp.float32),
                   jax.ShapeDtypeStruct((QROWS, 4 * DIM), jnp.float32)),
    )(*([ent_real.T] * 4), *([ent_imag.T] * 4))

    mesh = plsc.VectorSubcoreMesh(core_axis_name="c", subcore_axis_name="s",
                                  num_cores=NC, num_subcores=NS)
    partials = pl.kernel(
        _sc_body,
        out_type=jax.ShapeDtypeStruct((NW * L,), jnp.float32),
        mesh=mesh,
        compiler_params=pltpu.CompilerParams(needs_layout_passes=False,
                                             use_tc_tiling_on_sc=True),
        scratch_types=[
            pltpu.VMEM((NUM_RELS * DIM,), jnp.float32),
            pltpu.VMEM((NUM_RELS * DIM,), jnp.float32),
            pltpu.VMEM((NEG2,), i32),
            pltpu.VMEM((NEG2,), i32),
            pltpu.VMEM((NEG2,), i32),
            pltpu.VMEM((NEG2,), i32),
            pltpu.VMEM((NEG2,), i32),
            pltpu.VMEM((BPW,), i32),
            pltpu.VMEM((BPW,), i32),
            pltpu.VMEM((BPW,), i32),
            pltpu.VMEM((BPW,), i32),
            pltpu.VMEM((BPW,), i32),
            pltpu.VMEM((BPW,), jnp.float32),
            pltpu.VMEM((2 * L,), jnp.float32),
            pltpu.VMEM((CHUNK, 4 * DIM), jnp.float32),
            pltpu.VMEM((CHUNK, 4 * DIM), jnp.float32),
            pltpu.VMEM((CHUNK, 4 * DIM), jnp.float32),
            pltpu.VMEM((CHUNK, 4 * DIM), jnp.float32),
            pltpu.VMEM((CHUNK, 4 * DIM), jnp.float32),
            pltpu.VMEM((CHUNK, 4 * DIM), jnp.float32),
            pltpu.VMEM((CHUNK, 4 * DIM), jnp.float32),
            pltpu.VMEM((CHUNK, 4 * DIM), jnp.float32),
            pltpu.VMEM((L,), jnp.float32),
            pltpu.VMEM((L,), jnp.float32),
            pltpu.SemaphoreType.DMA,
            pltpu.SemaphoreType.DMA,
        ],
    )(
        hh // QROWS, tt // QROWS, r.astype(i32), w.astype(jnp.float32),
        a_flat // QROWS, nh_flat // QROWS, b_flat // QROWS, ntn_flat // QROWS,
        n_rel_hn.reshape(-1).astype(i32), n_rel_tn.reshape(-1).astype(i32),
        a_flat % QROWS, nh_flat % QROWS, b_flat % QROWS, ntn_flat % QROWS,
        hh % QROWS, tt % QROWS,
        er4, ei4,
        cos_t.reshape(-1), sin_t.reshape(-1), ab,
    )

    loss = pl.pallas_call(
        _fin_body,
        out_shape=jax.ShapeDtypeStruct((1, 1), jnp.float32),
        out_specs=pl.BlockSpec(memory_space=pltpu.SMEM),
    )(partials.reshape(NW, L))
    return loss[0, 0]


# speedup vs baseline: 2.4499x; 1.0821x over previous
import jax
import jax.numpy as jnp
from jax import lax
from jax.experimental import pallas as pl
from jax.experimental.pallas import tpu as pltpu
from jax.experimental.pallas import tpu_sc as plsc

NUM_CONS = 1000000
NUM_RELS = 1000
DIM = 32
BATCH = 4096
NEG = 20
REG_SCALE = 0.0001
MARGIN = 2.0
EMB_RANGE = (MARGIN + 2.0) / DIM
PI = 3.141592653589793

L = 16
NC, NS = 2, 16
NW = NC * NS
BPW = BATCH // NW
NEG_PER_W = BPW * NEG
NEG2 = 2 * NEG_PER_W
CHUNK = 64
NCHUNK = NEG2 // CHUNK
GROUPS = CHUNK // L
PREP_B = 4096
PREP_G = 62
QROWS = PREP_B * PREP_G


def _sigmoid(z):
    return 1.0 / (1.0 + jnp.exp(-z))


def _sc_body(h_hbm, t_hbm, r_hbm, w_hbm,
             a_hn_hbm, a_nh_hbm, b_nt_hbm, b_tn_hbm, r_hn_hbm, r_tn_hbm,
             ent_r_hbm, ent_i_hbm, cos_hbm, sin_hbm, ab_hbm,
             out_hbm,
             cos_v, sin_v, aidx_v, bidx_v, ridx_v,
             hidx_v, tidx_v, pridx_v, w_v, ab_v,
             e0, e1, e2, e3, f0, f1, f2, f3, srow, out_v, sem, sem2):
    wid = lax.axis_index("s") * NC + lax.axis_index("c")
    base = wid * BPW
    nb = wid * NEG_PER_W

    stage = [
        (cos_hbm, cos_v),
        (sin_hbm, sin_v),
        (a_hn_hbm.at[pl.ds(nb, NEG_PER_W)], aidx_v.at[pl.ds(0, NEG_PER_W)]),
        (a_nh_hbm.at[pl.ds(nb, NEG_PER_W)], aidx_v.at[pl.ds(NEG_PER_W, NEG_PER_W)]),
        (b_nt_hbm.at[pl.ds(nb, NEG_PER_W)], bidx_v.at[pl.ds(0, NEG_PER_W)]),
        (b_tn_hbm.at[pl.ds(nb, NEG_PER_W)], bidx_v.at[pl.ds(NEG_PER_W, NEG_PER_W)]),
        (r_hn_hbm.at[pl.ds(nb, NEG_PER_W)], ridx_v.at[pl.ds(0, NEG_PER_W)]),
        (r_tn_hbm.at[pl.ds(nb, NEG_PER_W)], ridx_v.at[pl.ds(NEG_PER_W, NEG_PER_W)]),
        (h_hbm.at[pl.ds(base, BPW)], hidx_v),
        (t_hbm.at[pl.ds(base, BPW)], tidx_v),
        (r_hbm.at[pl.ds(base, BPW)], pridx_v),
        (w_hbm.at[pl.ds(base, BPW)], w_v),
        (ab_hbm, ab_v),
    ]
    handles = [pltpu.async_copy(s, d, sem) for s, d in stage]
    for hnd in handles:
        hnd.wait()

    av = ab_v[pl.ds(0, L)]
    bv = ab_v[pl.ds(L, L)]
    zeros = jnp.zeros((L,), jnp.float32)
    lastmask = lax.iota(jnp.int32, L) == (L - 1)

    def score16(bufs, rowbase, rvec, wants_reg, regacc):
        b0, b1, b2, b3 = bufs
        for i in range(L):
            rb = rowbase + i
            hr0 = b0[rb, pl.ds(0, L)]
            hr1 = b0[rb, pl.ds(L, L)]
            hi0 = b1[rb, pl.ds(0, L)]
            hi1 = b1[rb, pl.ds(L, L)]
            tr0 = b2[rb, pl.ds(0, L)]
            tr1 = b2[rb, pl.ds(L, L)]
            ti0 = b3[rb, pl.ds(0, L)]
            ti1 = b3[rb, pl.ds(L, L)]
            rid = rvec[i]
            toff = rid * DIM
            cc0 = cos_v[pl.ds(toff, L)]
            cc1 = cos_v[pl.ds(toff + L, L)]
            ss0 = sin_v[pl.ds(toff, L)]
            ss1 = sin_v[pl.ds(toff + L, L)]
            u0 = hr0 * cc0 - hi0 * ss0 - tr0
            v0 = hr0 * ss0 + hi0 * cc0 - ti0
            u1 = hr1 * cc1 - hi1 * ss1 - tr1
            v1 = hr1 * ss1 + hi1 * cc1 - ti1
            acc = (u0 * u0 + v0 * v0) + (u1 * u1 + v1 * v1)
            c = plsc.cumsum(acc)
            plsc.store_scatter(srow, [jnp.full((L,), i, jnp.int32)], c,
                               mask=lastmask)
            if wants_reg:
                regacc = regacc + (hr0 * hr0 + hr1 * hr1 + hi0 * hi0
                                   + hi1 * hi1 + tr0 * tr0 + tr1 * tr1
                                   + ti0 * ti0 + ti1 * ti1 + cc0 * cc0
                                   + cc1 * cc1 + ss0 * ss0 + ss1 * ss1)
        sv = srow[...]
        p = _sigmoid(av * (MARGIN - sv) + bv)
        return p, regacc

    ebufs = (e0, e1, e2, e3)

    pacc = zeros
    regacc = zeros
    for k in range(2):
        gs = [
            pltpu.async_copy(ent_r_hbm.at[hidx_v.at[pl.ds(k * CHUNK, CHUNK)]], e0, sem),
            pltpu.async_copy(ent_i_hbm.at[hidx_v.at[pl.ds(k * CHUNK, CHUNK)]], e1, sem),
            pltpu.async_copy(ent_r_hbm.at[tidx_v.at[pl.ds(k * CHUNK, CHUNK)]], e2, sem),
            pltpu.async_copy(ent_i_hbm.at[tidx_v.at[pl.ds(k * CHUNK, CHUNK)]], e3, sem),
        ]
        for hnd in gs:
            hnd.wait()

        def pos_group(g, carry, _k=k):
            pacc, regacc = carry
            o = _k * CHUNK + g * L
            p, regacc = score16(ebufs, g * L, pridx_v[pl.ds(o, L)],
                                True, regacc)
            wv = w_v[pl.ds(o, L)]
            dlt = p - wv
            return pacc + dlt * dlt, regacc

        pacc, regacc = lax.fori_loop(0, GROUPS, pos_group, (pacc, regacc))

    slots = ((e0, e1, e2, e3, sem), (f0, f1, f2, f3, sem2))

    def fire(c, slot):
        b0, b1, b2, b3, sm = slot
        avw = aidx_v.at[pl.ds(c * CHUNK, CHUNK)]
        bvw = bidx_v.at[pl.ds(c * CHUNK, CHUNK)]
        pltpu.async_copy(ent_r_hbm.at[avw], b0, sm)
        pltpu.async_copy(ent_i_hbm.at[avw], b1, sm)
        pltpu.async_copy(ent_r_hbm.at[bvw], b2, sm)
        pltpu.async_copy(ent_i_hbm.at[bvw], b3, sm)

    def drain(slot):
        b0, b1, b2, b3, sm = slot
        for b in (b0, b1, b2, b3):
            pltpu.make_async_copy(ent_r_hbm.at[pl.ds(0, CHUNK)], b, sm).wait()

    def compute(c, slot, nacc):
        def grp(g, acc):
            o = c * CHUNK + g * L
            p, _ = score16(slot[:4], g * L, ridx_v[pl.ds(o, L)], False, None)
            return acc + p * p
        return lax.fori_loop(0, GROUPS, grp, nacc)

    fire(0, slots[0])
    fire(1, slots[1])

    def neg_pair(c2, nacc):
        c = c2 * 2
        drain(slots[0])
        nacc = compute(c, slots[0], nacc)

        @pl.when(c + 2 < NCHUNK)
        def _():
            fire(c + 2, slots[0])

        drain(slots[1])
        nacc = compute(c + 1, slots[1], nacc)

        @pl.when(c + 3 < NCHUNK)
        def _():
            fire(c + 3, slots[1])

        return nacc

    nacc = lax.fori_loop(0, NCHUNK // 2, neg_pair, zeros)

    part = (pacc * (1.0 / BATCH)
            + nacc * (1.0 / (2.0 * NEG * BATCH))
            + regacc * (REG_SCALE / (2.0 * BATCH)))
    out_v[...] = part
    pltpu.sync_copy(out_v, out_hbm.at[pl.ds(wid * L, L)])


def _trig_body(rel_ref, cos_ref, sin_ref):
    x = rel_ref[...] * (PI / EMB_RANGE)
    cos_ref[...] = jnp.cos(x)
    sin_ref[...] = jnp.sin(x)


def _prep_body(r0, r1, r2, r3, i0, i1, i2, i3, or_ref, oi_ref):
    or_ref[...] = jnp.concatenate(
        [r0[...].T, r1[...].T, r2[...].T, r3[...].T], axis=1)
    oi_ref[...] = jnp.concatenate(
        [i0[...].T, i1[...].T, i2[...].T, i3[...].T], axis=1)


def _fin_body(p_ref, o_ref):
    o_ref[0, 0] = jnp.sum(p_ref[...])


def kernel(h, r, t, w, n_hn, n_rel_hn, n_t, n_h, n_rel_tn, n_tn,
           s_h, s_r, s_t, s_w, ent_real, ent_imag, rel, lin_w, lin_b):
    del s_h, s_r, s_t, s_w
    i32 = jnp.int32
    cos_t, sin_t = pl.pallas_call(
        _trig_body,
        out_shape=(jax.ShapeDtypeStruct((NUM_RELS, DIM), jnp.float32),
                   jax.ShapeDtypeStruct((NUM_RELS, DIM), jnp.float32)),
    )(rel)

    def packed_row(x):
        x = x.astype(i32)
        return 4 * (x % QROWS) + x // QROWS

    hh = packed_row(h)
    tt = packed_row(t)
    a_flat = packed_row(n_hn.reshape(-1))
    nh_flat = packed_row(n_h.reshape(-1))
    b_flat = packed_row(n_t.reshape(-1))
    ntn_flat = packed_row(n_tn.reshape(-1))
    ab = jnp.concatenate([
        jnp.broadcast_to(lin_w.reshape(()), (L,)),
        jnp.broadcast_to(lin_b.reshape(()), (L,)),
    ]).astype(jnp.float32)

    maxb = NUM_CONS // PREP_B
    qspecs = [pl.BlockSpec((DIM, PREP_B),
                           lambda c, _j=j: (0, jnp.minimum(_j * PREP_G + c, maxb)))
              for j in range(4)]
    er4, ei4 = pl.pallas_call(
        _prep_body,
        grid=(PREP_G,),
        in_specs=qspecs + qspecs,
        out_specs=[pl.BlockSpec((PREP_B, 4 * DIM), lambda c: (c, 0)),
                   pl.BlockSpec((PREP_B, 4 * DIM), lambda c: (c, 0))],
        out_shape=(jax.ShapeDtypeStruct((QROWS, 4 * DIM), jnp.float32),
                   jax.ShapeDtypeStruct((QROWS, 4 * DIM), jnp.float32)),
    )(*([ent_real.T] * 4), *([ent_imag.T] * 4))
    er32 = er4.reshape(4 * QROWS, DIM)
    ei32 = ei4.reshape(4 * QROWS, DIM)

    mesh = plsc.VectorSubcoreMesh(core_axis_name="c", subcore_axis_name="s",
                                  num_cores=NC, num_subcores=NS)
    partials = pl.kernel(
        _sc_body,
        out_type=jax.ShapeDtypeStruct((NW * L,), jnp.float32),
        mesh=mesh,
        compiler_params=pltpu.CompilerParams(needs_layout_passes=False,
                                             use_tc_tiling_on_sc=False),
        scratch_types=[
            pltpu.VMEM((NUM_RELS * DIM,), jnp.float32),
            pltpu.VMEM((NUM_RELS * DIM,), jnp.float32),
            pltpu.VMEM((NEG2,), i32),
            pltpu.VMEM((NEG2,), i32),
            pltpu.VMEM((NEG2,), i32),
            pltpu.VMEM((BPW,), i32),
            pltpu.VMEM((BPW,), i32),
            pltpu.VMEM((BPW,), i32),
            pltpu.VMEM((BPW,), jnp.float32),
            pltpu.VMEM((2 * L,), jnp.float32),
            pltpu.VMEM((CHUNK, DIM), jnp.float32),
            pltpu.VMEM((CHUNK, DIM), jnp.float32),
            pltpu.VMEM((CHUNK, DIM), jnp.float32),
            pltpu.VMEM((CHUNK, DIM), jnp.float32),
            pltpu.VMEM((CHUNK, DIM), jnp.float32),
            pltpu.VMEM((CHUNK, DIM), jnp.float32),
            pltpu.VMEM((CHUNK, DIM), jnp.float32),
            pltpu.VMEM((CHUNK, DIM), jnp.float32),
            pltpu.VMEM((L,), jnp.float32),
            pltpu.VMEM((L,), jnp.float32),
            pltpu.SemaphoreType.DMA,
            pltpu.SemaphoreType.DMA,
        ],
    )(
        hh, tt, r.astype(i32), w.astype(jnp.float32),
        a_flat, nh_flat, b_flat, ntn_flat,
        n_rel_hn.reshape(-1).astype(i32), n_rel_tn.reshape(-1).astype(i32),
        er32, ei32,
        cos_t.reshape(-1), sin_t.reshape(-1), ab,
    )

    loss = pl.pallas_call(
        _fin_body,
        out_shape=jax.ShapeDtypeStruct((1, 1), jnp.float32),
        out_specs=pl.BlockSpec(memory_space=pltpu.SMEM),
    )(partials.reshape(NW, L))
    return loss[0, 0]


# speedup vs baseline: 3.6962x; 1.5087x over previous
import jax
import jax.numpy as jnp
from jax import lax
from jax.experimental import pallas as pl
from jax.experimental.pallas import tpu as pltpu
from jax.experimental.pallas import tpu_sc as plsc

NUM_CONS = 1000000
NUM_RELS = 1000
DIM = 32
BATCH = 4096
NEG = 20
REG_SCALE = 0.0001
MARGIN = 2.0
EMB_RANGE = (MARGIN + 2.0) / DIM
PI = 3.141592653589793

L = 16
NC, NS = 2, 16
NW = NC * NS
BPW = BATCH // NW
NEG_PER_W = BPW * NEG
NEG2 = 2 * NEG_PER_W
CHUNK = 64
NCHUNK = NEG2 // CHUNK
GROUPS = CHUNK // L
PREP_B = 4096
PREP_G = 62
QROWS = PREP_B * PREP_G


def _sigmoid(z):
    return 1.0 / (1.0 + jnp.exp(-z))


def _sc_body(h_hbm, t_hbm, r_hbm, w_hbm,
             a_hn_hbm, a_nh_hbm, b_nt_hbm, b_tn_hbm, r_hn_hbm, r_tn_hbm,
             ent_hbm, cos_hbm, sin_hbm, ab_hbm,
             out_hbm,
             cos_v, sin_v, aidx_v, bidx_v, ridx_v,
             hidx_v, tidx_v, pridx_v, w_v, ab_v,
             e0, e1, f0, f1, srow, out_v, sem, sem2):
    wid = lax.axis_index("s") * NC + lax.axis_index("c")
    base = wid * BPW
    nb = wid * NEG_PER_W

    stage = [
        (cos_hbm, cos_v),
        (sin_hbm, sin_v),
        (a_hn_hbm.at[pl.ds(nb, NEG_PER_W)], aidx_v.at[pl.ds(0, NEG_PER_W)]),
        (a_nh_hbm.at[pl.ds(nb, NEG_PER_W)], aidx_v.at[pl.ds(NEG_PER_W, NEG_PER_W)]),
        (b_nt_hbm.at[pl.ds(nb, NEG_PER_W)], bidx_v.at[pl.ds(0, NEG_PER_W)]),
        (b_tn_hbm.at[pl.ds(nb, NEG_PER_W)], bidx_v.at[pl.ds(NEG_PER_W, NEG_PER_W)]),
        (r_hn_hbm.at[pl.ds(nb, NEG_PER_W)], ridx_v.at[pl.ds(0, NEG_PER_W)]),
        (r_tn_hbm.at[pl.ds(nb, NEG_PER_W)], ridx_v.at[pl.ds(NEG_PER_W, NEG_PER_W)]),
        (h_hbm.at[pl.ds(base, BPW)], hidx_v),
        (t_hbm.at[pl.ds(base, BPW)], tidx_v),
        (r_hbm.at[pl.ds(base, BPW)], pridx_v),
        (w_hbm.at[pl.ds(base, BPW)], w_v),
        (ab_hbm, ab_v),
    ]
    handles = [pltpu.async_copy(s, d, sem) for s, d in stage]
    for hnd in handles:
        hnd.wait()

    av = ab_v[pl.ds(0, L)]
    bv = ab_v[pl.ds(L, L)]
    zeros = jnp.zeros((L,), jnp.float32)
    lastmask = lax.iota(jnp.int32, L) == (L - 1)

    def score16(bufs, rowbase, rvec, wants_reg, regacc):
        b0, b1 = bufs

        def unpk(buf, rb, half):
            w = plsc.bitcast(buf[rb, pl.ds(half * L, L)], jnp.bfloat16)
            return plsc.unpack(w, format=plsc.PackFormat.INTERLEAVED)

        for i in range(L):
            rb = rowbase + i
            hr0, hi0 = unpk(b0, rb, 0)
            hr1, hi1 = unpk(b0, rb, 1)
            tr0, ti0 = unpk(b1, rb, 0)
            tr1, ti1 = unpk(b1, rb, 1)
            rid = rvec[i]
            toff = rid * DIM
            cc0 = cos_v[pl.ds(toff, L)]
            cc1 = cos_v[pl.ds(toff + L, L)]
            ss0 = sin_v[pl.ds(toff, L)]
            ss1 = sin_v[pl.ds(toff + L, L)]
            u0 = hr0 * cc0 - hi0 * ss0 - tr0
            v0 = hr0 * ss0 + hi0 * cc0 - ti0
            u1 = hr1 * cc1 - hi1 * ss1 - tr1
            v1 = hr1 * ss1 + hi1 * cc1 - ti1
            acc = (u0 * u0 + v0 * v0) + (u1 * u1 + v1 * v1)
            c = plsc.cumsum(acc)
            plsc.store_scatter(srow, [jnp.full((L,), i, jnp.int32)], c,
                               mask=lastmask)
            if wants_reg:
                regacc = regacc + (hr0 * hr0 + hr1 * hr1 + hi0 * hi0
                                   + hi1 * hi1 + tr0 * tr0 + tr1 * tr1
                                   + ti0 * ti0 + ti1 * ti1 + cc0 * cc0
                                   + cc1 * cc1 + ss0 * ss0 + ss1 * ss1)
        sv = srow[...]
        p = _sigmoid(av * (MARGIN - sv) + bv)
        return p, regacc

    ebufs = (e0, e1)

    pacc = zeros
    regacc = zeros
    for k in range(2):
        gs = [
            pltpu.async_copy(ent_hbm.at[hidx_v.at[pl.ds(k * CHUNK, CHUNK)]], e0, sem),
            pltpu.async_copy(ent_hbm.at[tidx_v.at[pl.ds(k * CHUNK, CHUNK)]], e1, sem),
        ]
        for hnd in gs:
            hnd.wait()

        def pos_group(g, carry, _k=k):
            pacc, regacc = carry
            o = _k * CHUNK + g * L
            p, regacc = score16(ebufs, g * L, pridx_v[pl.ds(o, L)],
                                True, regacc)
            wv = w_v[pl.ds(o, L)]
            dlt = p - wv
            return pacc + dlt * dlt, regacc

        pacc, regacc = lax.fori_loop(0, GROUPS, pos_group, (pacc, regacc))

    slots = ((e0, e1, sem), (f0, f1, sem2))

    def fire(c, slot):
        b0, b1, sm = slot
        avw = aidx_v.at[pl.ds(c * CHUNK, CHUNK)]
        bvw = bidx_v.at[pl.ds(c * CHUNK, CHUNK)]
        pltpu.async_copy(ent_hbm.at[avw], b0, sm)
        pltpu.async_copy(ent_hbm.at[bvw], b1, sm)

    def drain(slot):
        b0, b1, sm = slot
        for b in (b0, b1):
            pltpu.make_async_copy(ent_hbm.at[pl.ds(0, CHUNK)], b, sm).wait()

    def compute(c, slot, nacc):
        def grp(g, acc):
            o = c * CHUNK + g * L
            p, _ = score16(slot[:2], g * L, ridx_v[pl.ds(o, L)], False, None)
            return acc + p * p
        return lax.fori_loop(0, GROUPS, grp, nacc)

    fire(0, slots[0])
    fire(1, slots[1])

    def neg_pair(c2, nacc):
        c = c2 * 2
        drain(slots[0])
        nacc = compute(c, slots[0], nacc)

        @pl.when(c + 2 < NCHUNK)
        def _():
            fire(c + 2, slots[0])

        drain(slots[1])
        nacc = compute(c + 1, slots[1], nacc)

        @pl.when(c + 3 < NCHUNK)
        def _():
            fire(c + 3, slots[1])

        return nacc

    nacc = lax.fori_loop(0, NCHUNK // 2, neg_pair, zeros)

    part = (pacc * (1.0 / BATCH)
            + nacc * (1.0 / (2.0 * NEG * BATCH))
            + regacc * (REG_SCALE / (2.0 * BATCH)))
    out_v[...] = part
    pltpu.sync_copy(out_v, out_hbm.at[pl.ds(wid * L, L)])


def _trig_body(rel_ref, cos_ref, sin_ref):
    x = rel_ref[...] * (PI / EMB_RANGE)
    cos_ref[...] = jnp.cos(x)
    sin_ref[...] = jnp.sin(x)


def _pk(r, i):
    ur = lax.bitcast_convert_type(r.astype(jnp.bfloat16), jnp.uint16)
    ui = lax.bitcast_convert_type(i.astype(jnp.bfloat16), jnp.uint16)
    w = ur.astype(jnp.uint32) | (ui.astype(jnp.uint32) << 16)
    return lax.bitcast_convert_type(w, jnp.float32)


def _prep_body(r0, r1, r2, r3, i0, i1, i2, i3, o_ref):
    o_ref[...] = jnp.concatenate(
        [_pk(r0[...], i0[...]).T, _pk(r1[...], i1[...]).T,
         _pk(r2[...], i2[...]).T, _pk(r3[...], i3[...]).T], axis=1)


def _fin_body(p_ref, o_ref):
    o_ref[0, 0] = jnp.sum(p_ref[...])


def kernel(h, r, t, w, n_hn, n_rel_hn, n_t, n_h, n_rel_tn, n_tn,
           s_h, s_r, s_t, s_w, ent_real, ent_imag, rel, lin_w, lin_b):
    del s_h, s_r, s_t, s_w
    i32 = jnp.int32
    cos_t, sin_t = pl.pallas_call(
        _trig_body,
        out_shape=(jax.ShapeDtypeStruct((NUM_RELS, DIM), jnp.float32),
                   jax.ShapeDtypeStruct((NUM_RELS, DIM), jnp.float32)),
    )(rel)

    def packed_row(x):
        x = x.astype(i32)
        return 4 * (x % QROWS) + x // QROWS

    hh = packed_row(h)
    tt = packed_row(t)
    a_flat = packed_row(n_hn.reshape(-1))
    nh_flat = packed_row(n_h.reshape(-1))
    b_flat = packed_row(n_t.reshape(-1))
    ntn_flat = packed_row(n_tn.reshape(-1))
    ab = jnp.concatenate([
        jnp.broadcast_to(lin_w.reshape(()), (L,)),
        jnp.broadcast_to(lin_b.reshape(()), (L,)),
    ]).astype(jnp.float32)

    maxb = NUM_CONS // PREP_B
    qspecs = [pl.BlockSpec((DIM, PREP_B),
                           lambda c, _j=j: (0, jnp.minimum(_j * PREP_G + c, maxb)))
              for j in range(4)]
    ec4 = pl.pallas_call(
        _prep_body,
        grid=(PREP_G,),
        in_specs=qspecs + qspecs,
        out_specs=pl.BlockSpec((PREP_B, 4 * DIM), lambda c: (c, 0)),
        out_shape=jax.ShapeDtypeStruct((QROWS, 4 * DIM), jnp.float32),
    )(*([ent_real.T] * 4), *([ent_imag.T] * 4))
    ec32 = ec4.reshape(4 * QROWS, DIM)

    mesh = plsc.VectorSubcoreMesh(core_axis_name="c", subcore_axis_name="s",
                                  num_cores=NC, num_subcores=NS)
    partials = pl.kernel(
        _sc_body,
        out_type=jax.ShapeDtypeStruct((NW * L,), jnp.float32),
        mesh=mesh,
        compiler_params=pltpu.CompilerParams(needs_layout_passes=False,
                                             use_tc_tiling_on_sc=False),
        scratch_types=[
            pltpu.VMEM((NUM_RELS * DIM,), jnp.float32),
            pltpu.VMEM((NUM_RELS * DIM,), jnp.float32),
            pltpu.VMEM((NEG2,), i32),
            pltpu.VMEM((NEG2,), i32),
            pltpu.VMEM((NEG2,), i32),
            pltpu.VMEM((BPW,), i32),
            pltpu.VMEM((BPW,), i32),
            pltpu.VMEM((BPW,), i32),
            pltpu.VMEM((BPW,), jnp.float32),
            pltpu.VMEM((2 * L,), jnp.float32),
            pltpu.VMEM((CHUNK, DIM), jnp.float32),
            pltpu.VMEM((CHUNK, DIM), jnp.float32),
            pltpu.VMEM((CHUNK, DIM), jnp.float32),
            pltpu.VMEM((CHUNK, DIM), jnp.float32),
            pltpu.VMEM((L,), jnp.float32),
            pltpu.VMEM((L,), jnp.float32),
            pltpu.SemaphoreType.DMA,
            pltpu.SemaphoreType.DMA,
        ],
    )(
        hh, tt, r.astype(i32), w.astype(jnp.float32),
        a_flat, nh_flat, b_flat, ntn_flat,
        n_rel_hn.reshape(-1).astype(i32), n_rel_tn.reshape(-1).astype(i32),
        ec32,
        cos_t.reshape(-1), sin_t.reshape(-1), ab,
    )

    loss = pl.pallas_call(
        _fin_body,
        out_shape=jax.ShapeDtypeStruct((1, 1), jnp.float32),
        out_specs=pl.BlockSpec(memory_space=pltpu.SMEM),
    )(partials.reshape(NW, L))
    return loss[0, 0]


# speedup vs baseline: 3.7482x; 1.0141x over previous
import jax
import jax.numpy as jnp
from jax import lax
from jax.experimental import pallas as pl
from jax.experimental.pallas import tpu as pltpu
from jax.experimental.pallas import tpu_sc as plsc

NUM_CONS = 1000000
NUM_RELS = 1000
DIM = 32
BATCH = 4096
NEG = 20
REG_SCALE = 0.0001
MARGIN = 2.0
EMB_RANGE = (MARGIN + 2.0) / DIM
PI = 3.141592653589793

L = 16
NC, NS = 2, 16
NW = NC * NS
BPW = BATCH // NW
NEG_PER_W = BPW * NEG
NEG2 = 2 * NEG_PER_W
CHUNK = 128
NCHUNK = NEG2 // CHUNK
GROUPS = CHUNK // L
PREP_B = 8192
PREP_G = 31
QROWS = PREP_B * PREP_G


def _sigmoid(z):
    return 1.0 / (1.0 + jnp.exp(-z))


def _sc_body(h_hbm, t_hbm, r_hbm, w_hbm,
             a_hn_hbm, a_nh_hbm, b_nt_hbm, b_tn_hbm, r_hn_hbm, r_tn_hbm,
             ent_hbm, cos_hbm, sin_hbm, ab_hbm,
             out_hbm,
             cos_v, sin_v, aidx_v, bidx_v, ridx_v,
             hidx_v, tidx_v, pridx_v, w_v, ab_v,
             e0, e1, f0, f1, srow, out_v, sem, sem2):
    wid = lax.axis_index("s") * NC + lax.axis_index("c")
    base = wid * BPW
    nb = wid * NEG_PER_W

    stage = [
        (cos_hbm, cos_v),
        (sin_hbm, sin_v),
        (a_hn_hbm.at[pl.ds(nb, NEG_PER_W)], aidx_v.at[pl.ds(0, NEG_PER_W)]),
        (a_nh_hbm.at[pl.ds(nb, NEG_PER_W)], aidx_v.at[pl.ds(NEG_PER_W, NEG_PER_W)]),
        (b_nt_hbm.at[pl.ds(nb, NEG_PER_W)], bidx_v.at[pl.ds(0, NEG_PER_W)]),
        (b_tn_hbm.at[pl.ds(nb, NEG_PER_W)], bidx_v.at[pl.ds(NEG_PER_W, NEG_PER_W)]),
        (r_hn_hbm.at[pl.ds(nb, NEG_PER_W)], ridx_v.at[pl.ds(0, NEG_PER_W)]),
        (r_tn_hbm.at[pl.ds(nb, NEG_PER_W)], ridx_v.at[pl.ds(NEG_PER_W, NEG_PER_W)]),
        (h_hbm.at[pl.ds(base, BPW)], hidx_v),
        (t_hbm.at[pl.ds(base, BPW)], tidx_v),
        (r_hbm.at[pl.ds(base, BPW)], pridx_v),
        (w_hbm.at[pl.ds(base, BPW)], w_v),
        (ab_hbm, ab_v),
    ]
    handles = [pltpu.async_copy(s, d, sem) for s, d in stage]
    for hnd in handles:
        hnd.wait()

    av = ab_v[pl.ds(0, L)]
    bv = ab_v[pl.ds(L, L)]
    zeros = jnp.zeros((L,), jnp.float32)
    lastmask = lax.iota(jnp.int32, L) == (L - 1)

    def score16(bufs, rowbase, rvec, wants_reg, regacc):
        b0, b1 = bufs

        def unpk(buf, rb, half):
            w = plsc.bitcast(buf[rb, pl.ds(half * L, L)], jnp.bfloat16)
            return plsc.unpack(w, format=plsc.PackFormat.INTERLEAVED)

        for i in range(L):
            rb = rowbase + i
            hr0, hi0 = unpk(b0, rb, 0)
            hr1, hi1 = unpk(b0, rb, 1)
            tr0, ti0 = unpk(b1, rb, 0)
            tr1, ti1 = unpk(b1, rb, 1)
            rid = rvec[i]
            toff = rid * DIM
            cc0 = cos_v[pl.ds(toff, L)]
            cc1 = cos_v[pl.ds(toff + L, L)]
            ss0 = sin_v[pl.ds(toff, L)]
            ss1 = sin_v[pl.ds(toff + L, L)]
            u0 = hr0 * cc0 - hi0 * ss0 - tr0
            v0 = hr0 * ss0 + hi0 * cc0 - ti0
            u1 = hr1 * cc1 - hi1 * ss1 - tr1
            v1 = hr1 * ss1 + hi1 * cc1 - ti1
            acc = (u0 * u0 + v0 * v0) + (u1 * u1 + v1 * v1)
            c = plsc.cumsum(acc)
            plsc.store_scatter(srow, [jnp.full((L,), i, jnp.int32)], c,
                               mask=lastmask)
            if wants_reg:
                regacc = regacc + (hr0 * hr0 + hr1 * hr1 + hi0 * hi0
                                   + hi1 * hi1 + tr0 * tr0 + tr1 * tr1
                                   + ti0 * ti0 + ti1 * ti1 + cc0 * cc0
                                   + cc1 * cc1 + ss0 * ss0 + ss1 * ss1)
        sv = srow[...]
        p = _sigmoid(av * (MARGIN - sv) + bv)
        return p, regacc

    ebufs = (e0, e1)

    pacc = zeros
    regacc = zeros
    for k in range(BPW // CHUNK):
        gs = [
            pltpu.async_copy(ent_hbm.at[hidx_v.at[pl.ds(k * CHUNK, CHUNK)]], e0, sem),
            pltpu.async_copy(ent_hbm.at[tidx_v.at[pl.ds(k * CHUNK, CHUNK)]], e1, sem),
        ]
        for hnd in gs:
            hnd.wait()

        def pos_group(g, carry, _k=k):
            pacc, regacc = carry
            o = _k * CHUNK + g * L
            p, regacc = score16(ebufs, g * L, pridx_v[pl.ds(o, L)],
                                True, regacc)
            wv = w_v[pl.ds(o, L)]
            dlt = p - wv
            return pacc + dlt * dlt, regacc

        pacc, regacc = lax.fori_loop(0, GROUPS, pos_group, (pacc, regacc))

    slots = ((e0, e1, sem), (f0, f1, sem2))

    def fire(c, slot):
        b0, b1, sm = slot
        avw = aidx_v.at[pl.ds(c * CHUNK, CHUNK)]
        bvw = bidx_v.at[pl.ds(c * CHUNK, CHUNK)]
        pltpu.async_copy(ent_hbm.at[avw], b0, sm)
        pltpu.async_copy(ent_hbm.at[bvw], b1, sm)

    def drain(slot):
        b0, b1, sm = slot
        for b in (b0, b1):
            pltpu.make_async_copy(ent_hbm.at[pl.ds(0, CHUNK)], b, sm).wait()

    def compute(c, slot, nacc):
        def grp(g, acc):
            o = c * CHUNK + g * L
            p, _ = score16(slot[:2], g * L, ridx_v[pl.ds(o, L)], False, None)
            return acc + p * p
        return lax.fori_loop(0, GROUPS, grp, nacc)

    fire(0, slots[0])
    fire(1, slots[1])

    def neg_pair(c2, nacc):
        c = c2 * 2
        drain(slots[0])
        nacc = compute(c, slots[0], nacc)

        @pl.when(c + 2 < NCHUNK)
        def _():
            fire(c + 2, slots[0])

        drain(slots[1])
        nacc = compute(c + 1, slots[1], nacc)

        @pl.when(c + 3 < NCHUNK)
        def _():
            fire(c + 3, slots[1])

        return nacc

    nacc = lax.fori_loop(0, NCHUNK // 2, neg_pair, zeros)

    part = (pacc * (1.0 / BATCH)
            + nacc * (1.0 / (2.0 * NEG * BATCH))
            + regacc * (REG_SCALE / (2.0 * BATCH)))
    out_v[...] = part
    pltpu.sync_copy(out_v, out_hbm.at[pl.ds(wid * L, L)])


def _trig_body(rel_ref, cos_ref, sin_ref):
    x = rel_ref[...] * (PI / EMB_RANGE)
    cos_ref[...] = jnp.cos(x)
    sin_ref[...] = jnp.sin(x)


def _pk(r, i):
    ur = lax.bitcast_convert_type(r.astype(jnp.bfloat16), jnp.uint16)
    ui = lax.bitcast_convert_type(i.astype(jnp.bfloat16), jnp.uint16)
    w = ur.astype(jnp.uint32) | (ui.astype(jnp.uint32) << 16)
    return lax.bitcast_convert_type(w, jnp.float32)


def _prep_body(r0, r1, r2, r3, i0, i1, i2, i3, o_ref):
    o_ref[...] = jnp.concatenate(
        [_pk(r0[...], i0[...]).T, _pk(r1[...], i1[...]).T,
         _pk(r2[...], i2[...]).T, _pk(r3[...], i3[...]).T], axis=1)


def _fin_body(p_ref, o_ref):
    o_ref[0, 0] = jnp.sum(p_ref[...])


def kernel(h, r, t, w, n_hn, n_rel_hn, n_t, n_h, n_rel_tn, n_tn,
           s_h, s_r, s_t, s_w, ent_real, ent_imag, rel, lin_w, lin_b):
    del s_h, s_r, s_t, s_w
    i32 = jnp.int32
    cos_t, sin_t = pl.pallas_call(
        _trig_body,
        out_shape=(jax.ShapeDtypeStruct((NUM_RELS, DIM), jnp.float32),
                   jax.ShapeDtypeStruct((NUM_RELS, DIM), jnp.float32)),
    )(rel)

    def packed_row(x):
        x = x.astype(i32)
        return 4 * (x % QROWS) + x // QROWS

    hh = packed_row(h)
    tt = packed_row(t)
    a_flat = packed_row(n_hn.reshape(-1))
    nh_flat = packed_row(n_h.reshape(-1))
    b_flat = packed_row(n_t.reshape(-1))
    ntn_flat = packed_row(n_tn.reshape(-1))
    ab = jnp.concatenate([
        jnp.broadcast_to(lin_w.reshape(()), (L,)),
        jnp.broadcast_to(lin_b.reshape(()), (L,)),
    ]).astype(jnp.float32)

    maxb = NUM_CONS // PREP_B
    qspecs = [pl.BlockSpec((DIM, PREP_B),
                           lambda c, _j=j: (0, jnp.minimum(_j * PREP_G + c, maxb)))
              for j in range(4)]
    ec4 = pl.pallas_call(
        _prep_body,
        grid=(PREP_G,),
        in_specs=qspecs + qspecs,
        out_specs=pl.BlockSpec((PREP_B, 4 * DIM), lambda c: (c, 0)),
        out_shape=jax.ShapeDtypeStruct((QROWS, 4 * DIM), jnp.float32),
    )(*([ent_real.T] * 4), *([ent_imag.T] * 4))
    ec32 = ec4.reshape(4 * QROWS, DIM)

    mesh = plsc.VectorSubcoreMesh(core_axis_name="c", subcore_axis_name="s",
                                  num_cores=NC, num_subcores=NS)
    partials = pl.kernel(
        _sc_body,
        out_type=jax.ShapeDtypeStruct((NW * L,), jnp.float32),
        mesh=mesh,
        compiler_params=pltpu.CompilerParams(needs_layout_passes=False,
                                             use_tc_tiling_on_sc=False),
        scratch_types=[
            pltpu.VMEM((NUM_RELS * DIM,), jnp.float32),
            pltpu.VMEM((NUM_RELS * DIM,), jnp.float32),
            pltpu.VMEM((NEG2,), i32),
            pltpu.VMEM((NEG2,), i32),
            pltpu.VMEM((NEG2,), i32),
            pltpu.VMEM((BPW,), i32),
            pltpu.VMEM((BPW,), i32),
            pltpu.VMEM((BPW,), i32),
            pltpu.VMEM((BPW,), jnp.float32),
            pltpu.VMEM((2 * L,), jnp.float32),
            pltpu.VMEM((CHUNK, DIM), jnp.float32),
            pltpu.VMEM((CHUNK, DIM), jnp.float32),
            pltpu.VMEM((CHUNK, DIM), jnp.float32),
            pltpu.VMEM((CHUNK, DIM), jnp.float32),
            pltpu.VMEM((L,), jnp.float32),
            pltpu.VMEM((L,), jnp.float32),
            pltpu.SemaphoreType.DMA,
            pltpu.SemaphoreType.DMA,
        ],
    )(
        hh, tt, r.astype(i32), w.astype(jnp.float32),
        a_flat, nh_flat, b_flat, ntn_flat,
        n_rel_hn.reshape(-1).astype(i32), n_rel_tn.reshape(-1).astype(i32),
        ec32,
        cos_t.reshape(-1), sin_t.reshape(-1), ab,
    )

    loss = pl.pallas_call(
        _fin_body,
        out_shape=jax.ShapeDtypeStruct((1, 1), jnp.float32),
        out_specs=pl.BlockSpec(memory_space=pltpu.SMEM),
    )(partials.reshape(NW, L))
    return loss[0, 0]


# speedup vs baseline: 4.0733x; 1.0867x over previous
import jax
import jax.numpy as jnp
from jax import lax
from jax.experimental import pallas as pl
from jax.experimental.pallas import tpu as pltpu
from jax.experimental.pallas import tpu_sc as plsc

NUM_CONS = 1000000
NUM_RELS = 1000
DIM = 32
BATCH = 4096
NEG = 20
REG_SCALE = 0.0001
MARGIN = 2.0
EMB_RANGE = (MARGIN + 2.0) / DIM
PI = 3.141592653589793

L = 16
NC, NS = 2, 16
NW = NC * NS
BPW = BATCH // NW
NEG_PER_W = BPW * NEG
NEG2 = 2 * NEG_PER_W
CHUNK = 128
NCHUNK = NEG2 // CHUNK
GROUPS = CHUNK // L
PREP_B = 8192
PREP_G = 31
QROWS = PREP_B * PREP_G


def _sigmoid(z):
    return 1.0 / (1.0 + jnp.exp(-z))


def _sc_body(pidx_hbm, relidx_hbm, w_hbm,
             ent_hbm, cos_hbm, sin_hbm, ab_hbm,
             out_hbm,
             cos_v, sin_v, aidx_v, bidx_v, ridx_v,
             hidx_v, tidx_v, pridx_v, w_v, ab_v,
             e0, e1, f0, f1, srow, out_v, sem, sem2):
    wid = lax.axis_index("s") * NC + lax.axis_index("c")
    base = wid * BPW
    nb = wid * NEG_PER_W

    NA = BATCH * NEG
    stage = [
        (cos_hbm, cos_v),
        (sin_hbm, sin_v),
        (pidx_hbm.at[pl.ds(2 * BATCH + nb, NEG_PER_W)], aidx_v.at[pl.ds(0, NEG_PER_W)]),
        (pidx_hbm.at[pl.ds(2 * BATCH + NA + nb, NEG_PER_W)], aidx_v.at[pl.ds(NEG_PER_W, NEG_PER_W)]),
        (pidx_hbm.at[pl.ds(2 * BATCH + 2 * NA + nb, NEG_PER_W)], bidx_v.at[pl.ds(0, NEG_PER_W)]),
        (pidx_hbm.at[pl.ds(2 * BATCH + 3 * NA + nb, NEG_PER_W)], bidx_v.at[pl.ds(NEG_PER_W, NEG_PER_W)]),
        (relidx_hbm.at[pl.ds(BATCH + nb, NEG_PER_W)], ridx_v.at[pl.ds(0, NEG_PER_W)]),
        (relidx_hbm.at[pl.ds(BATCH + NA + nb, NEG_PER_W)], ridx_v.at[pl.ds(NEG_PER_W, NEG_PER_W)]),
        (pidx_hbm.at[pl.ds(base, BPW)], hidx_v),
        (pidx_hbm.at[pl.ds(BATCH + base, BPW)], tidx_v),
        (relidx_hbm.at[pl.ds(base, BPW)], pridx_v),
        (w_hbm.at[pl.ds(base, BPW)], w_v),
        (ab_hbm, ab_v),
    ]
    handles = [pltpu.async_copy(s, d, sem) for s, d in stage]
    for hnd in handles:
        hnd.wait()

    av = ab_v[pl.ds(0, L)]
    bv = ab_v[pl.ds(L, L)]
    zeros = jnp.zeros((L,), jnp.float32)
    lastmask = lax.iota(jnp.int32, L) == (L - 1)

    def score16(bufs, rowbase, rvec, wants_reg, regacc):
        b0, b1 = bufs

        def unpk(buf, rb, half):
            w = plsc.bitcast(buf[rb, pl.ds(half * L, L)], jnp.bfloat16)
            return plsc.unpack(w, format=plsc.PackFormat.INTERLEAVED)

        for i in range(L):
            rb = rowbase + i
            hr0, hi0 = unpk(b0, rb, 0)
            hr1, hi1 = unpk(b0, rb, 1)
            tr0, ti0 = unpk(b1, rb, 0)
            tr1, ti1 = unpk(b1, rb, 1)
            rid = rvec[i]
            toff = rid * DIM
            cc0 = cos_v[pl.ds(toff, L)]
            cc1 = cos_v[pl.ds(toff + L, L)]
            ss0 = sin_v[pl.ds(toff, L)]
            ss1 = sin_v[pl.ds(toff + L, L)]
            u0 = hr0 * cc0 - hi0 * ss0 - tr0
            v0 = hr0 * ss0 + hi0 * cc0 - ti0
            u1 = hr1 * cc1 - hi1 * ss1 - tr1
            v1 = hr1 * ss1 + hi1 * cc1 - ti1
            acc = (u0 * u0 + v0 * v0) + (u1 * u1 + v1 * v1)
            c = plsc.cumsum(acc)
            plsc.store_scatter(srow, [jnp.full((L,), i, jnp.int32)], c,
                               mask=lastmask)
            if wants_reg:
                regacc = regacc + (hr0 * hr0 + hr1 * hr1 + hi0 * hi0
                                   + hi1 * hi1 + tr0 * tr0 + tr1 * tr1
                                   + ti0 * ti0 + ti1 * ti1 + cc0 * cc0
                                   + cc1 * cc1 + ss0 * ss0 + ss1 * ss1)
        sv = srow[...]
        p = _sigmoid(av * (MARGIN - sv) + bv)
        return p, regacc

    ebufs = (e0, e1)

    pacc = zeros
    regacc = zeros
    for k in range(BPW // CHUNK):
        gs = [
            pltpu.async_copy(ent_hbm.at[hidx_v.at[pl.ds(k * CHUNK, CHUNK)]], e0, sem),
            pltpu.async_copy(ent_hbm.at[tidx_v.at[pl.ds(k * CHUNK, CHUNK)]], e1, sem),
        ]
        for hnd in gs:
            hnd.wait()

        def pos_group(g, carry, _k=k):
            pacc, regacc = carry
            o = _k * CHUNK + g * L
            p, regacc = score16(ebufs, g * L, pridx_v[pl.ds(o, L)],
                                True, regacc)
            wv = w_v[pl.ds(o, L)]
            dlt = p - wv
            return pacc + dlt * dlt, regacc

        pacc, regacc = lax.fori_loop(0, GROUPS, pos_group, (pacc, regacc))

    slots = ((e0, e1, sem), (f0, f1, sem2))

    def fire(c, slot):
        b0, b1, sm = slot
        avw = aidx_v.at[pl.ds(c * CHUNK, CHUNK)]
        bvw = bidx_v.at[pl.ds(c * CHUNK, CHUNK)]
        pltpu.async_copy(ent_hbm.at[avw], b0, sm)
        pltpu.async_copy(ent_hbm.at[bvw], b1, sm)

    def drain(slot):
        b0, b1, sm = slot
        for b in (b0, b1):
            pltpu.make_async_copy(ent_hbm.at[pl.ds(0, CHUNK)], b, sm).wait()

    def compute(c, slot, nacc):
        def grp(g, acc):
            o = c * CHUNK + g * L
            p, _ = score16(slot[:2], g * L, ridx_v[pl.ds(o, L)], False, None)
            return acc + p * p
        return lax.fori_loop(0, GROUPS, grp, nacc)

    fire(0, slots[0])
    fire(1, slots[1])

    def neg_pair(c2, nacc):
        c = c2 * 2
        drain(slots[0])
        nacc = compute(c, slots[0], nacc)

        @pl.when(c + 2 < NCHUNK)
        def _():
            fire(c + 2, slots[0])

        drain(slots[1])
        nacc = compute(c + 1, slots[1], nacc)

        @pl.when(c + 3 < NCHUNK)
        def _():
            fire(c + 3, slots[1])

        return nacc

    nacc = lax.fori_loop(0, NCHUNK // 2, neg_pair, zeros)

    part = (pacc * (1.0 / BATCH)
            + nacc * (1.0 / (2.0 * NEG * BATCH))
            + regacc * (REG_SCALE / (2.0 * BATCH)))
    out_v[...] = part
    pltpu.sync_copy(out_v, out_hbm.at[pl.ds(wid * L, L)])


def _trig_body(rel_ref, cos_ref, sin_ref):
    x = rel_ref[...] * (PI / EMB_RANGE)
    cos_ref[...] = jnp.cos(x)
    sin_ref[...] = jnp.sin(x)


def _pk(r, i):
    ur = lax.bitcast_convert_type(r.astype(jnp.bfloat16), jnp.uint16)
    ui = lax.bitcast_convert_type(i.astype(jnp.bfloat16), jnp.uint16)
    w = ur.astype(jnp.uint32) | (ui.astype(jnp.uint32) << 16)
    return lax.bitcast_convert_type(w, jnp.float32)


def _prep_body(r0, r1, r2, r3, i0, i1, i2, i3, o_ref):
    o_ref[...] = jnp.concatenate(
        [_pk(r0[...], i0[...]).T, _pk(r1[...], i1[...]).T,
         _pk(r2[...], i2[...]).T, _pk(r3[...], i3[...]).T], axis=1)


def _fin_body(p_ref, o_ref):
    o_ref[0, 0] = jnp.sum(p_ref[...])


def kernel(h, r, t, w, n_hn, n_rel_hn, n_t, n_h, n_rel_tn, n_tn,
           s_h, s_r, s_t, s_w, ent_real, ent_imag, rel, lin_w, lin_b):
    del s_h, s_r, s_t, s_w
    i32 = jnp.int32
    cos_t, sin_t = pl.pallas_call(
        _trig_body,
        out_shape=(jax.ShapeDtypeStruct((NUM_RELS, DIM), jnp.float32),
                   jax.ShapeDtypeStruct((NUM_RELS, DIM), jnp.float32)),
    )(rel)

    def packed_row(x):
        x = x.astype(i32)
        return 4 * (x % QROWS) + x // QROWS

    pidx = packed_row(jnp.concatenate([
        h.astype(i32), t.astype(i32),
        n_hn.reshape(-1).astype(i32), n_h.reshape(-1).astype(i32),
        n_t.reshape(-1).astype(i32), n_tn.reshape(-1).astype(i32)]))
    relidx = jnp.concatenate([
        r.astype(i32), n_rel_hn.reshape(-1).astype(i32),
        n_rel_tn.reshape(-1).astype(i32)])
    ab = jnp.concatenate([
        jnp.broadcast_to(lin_w.reshape(()), (L,)),
        jnp.broadcast_to(lin_b.reshape(()), (L,)),
    ]).astype(jnp.float32)

    maxb = NUM_CONS // PREP_B
    qspecs = [pl.BlockSpec((DIM, PREP_B),
                           lambda c, _j=j: (0, jnp.minimum(_j * PREP_G + c, maxb)))
              for j in range(4)]
    ec4 = pl.pallas_call(
        _prep_body,
        grid=(PREP_G,),
        in_specs=qspecs + qspecs,
        out_specs=pl.BlockSpec((PREP_B, 4 * DIM), lambda c: (c, 0)),
        out_shape=jax.ShapeDtypeStruct((QROWS, 4 * DIM), jnp.float32),
    )(*([ent_real.T] * 4), *([ent_imag.T] * 4))
    ec32 = ec4.reshape(4 * QROWS, DIM)

    mesh = plsc.VectorSubcoreMesh(core_axis_name="c", subcore_axis_name="s",
                                  num_cores=NC, num_subcores=NS)
    partials = pl.kernel(
        _sc_body,
        out_type=jax.ShapeDtypeStruct((NW * L,), jnp.float32),
        mesh=mesh,
        compiler_params=pltpu.CompilerParams(needs_layout_passes=False,
                                             use_tc_tiling_on_sc=False),
        scratch_types=[
            pltpu.VMEM((NUM_RELS * DIM,), jnp.float32),
            pltpu.VMEM((NUM_RELS * DIM,), jnp.float32),
            pltpu.VMEM((NEG2,), i32),
            pltpu.VMEM((NEG2,), i32),
            pltpu.VMEM((NEG2,), i32),
            pltpu.VMEM((BPW,), i32),
            pltpu.VMEM((BPW,), i32),
            pltpu.VMEM((BPW,), i32),
            pltpu.VMEM((BPW,), jnp.float32),
            pltpu.VMEM((2 * L,), jnp.float32),
            pltpu.VMEM((CHUNK, DIM), jnp.float32),
            pltpu.VMEM((CHUNK, DIM), jnp.float32),
            pltpu.VMEM((CHUNK, DIM), jnp.float32),
            pltpu.VMEM((CHUNK, DIM), jnp.float32),
            pltpu.VMEM((L,), jnp.float32),
            pltpu.VMEM((L,), jnp.float32),
            pltpu.SemaphoreType.DMA,
            pltpu.SemaphoreType.DMA,
        ],
    )(
        pidx, relidx, w.astype(jnp.float32),
        ec32,
        cos_t.reshape(-1), sin_t.reshape(-1), ab,
    )

    loss = pl.pallas_call(
        _fin_body,
        out_shape=jax.ShapeDtypeStruct((1, 1), jnp.float32),
        out_specs=pl.BlockSpec(memory_space=pltpu.SMEM),
    )(partials.reshape(NW, L))
    return loss[0, 0]
